# baseline (speedup 1.0000x reference)
import functools

import jax
import jax.numpy as jnp
import numpy as np
from jax import lax
from jax.experimental import pallas as pl
from jax.experimental.pallas import tpu as pltpu
from jax.experimental.pallas import tpu_sc as plsc

N_DRUGS = 218
N_NODES = 489
BN_EPS = 1e-5
F32 = jnp.float32
BF16 = jnp.bfloat16

ND = 6540
NDP = 6656
ROWB = 512


def _pad2(a, r, c):
    return jnp.zeros((r, c), a.dtype).at[: a.shape[0], : a.shape[1]].set(a)


def _pad1(a, n):
    return jnp.zeros((n,), a.dtype).at[: a.shape[0]].set(a)


def _bdot(a, b):
    return jax.lax.dot(a.astype(BF16), b.astype(BF16),
                       preferred_element_type=F32)


def _drug_l1_body(x_ref, w_ref, dinv_ref, u_ref):
    u_ref[...] = dinv_ref[...][:, None] * _bdot(x_ref[...], w_ref[...])


def _drug_mid_body(agg_ref, u_ref, dinv_ref, b_ref, w_ref, uo_ref, x_ref):
    dinv = dinv_ref[...][:, None]
    x = jax.nn.relu(dinv * (agg_ref[...] + u_ref[...]) + b_ref[...][None, :])
    x_ref[...] = x
    uo_ref[...] = dinv * _bdot(x, w_ref[...])


def _drug_fin_body(agg_ref, u_ref, dinv_ref, b_ref, x_ref):
    dinv = dinv_ref[...][:, None]
    x_ref[...] = jax.nn.relu(dinv * (agg_ref[...] + u_ref[...])
                             + b_ref[...][None, :])


def _row_spec(c):
    return pl.BlockSpec((ROWB, c), lambda i: (i, 0))


def _vec_spec(n):
    return pl.BlockSpec((n,), lambda i: (0,))


def _full_spec(r, c):
    return pl.BlockSpec((r, c), lambda i: (0, 0))


def _drug_l1(x, w, dinv, fin, fout):
    return pl.pallas_call(
        _drug_l1_body,
        grid=(NDP // ROWB,),
        in_specs=[_row_spec(fin), _full_spec(fin, fout), pl.BlockSpec((ROWB,), lambda i: (i,))],
        out_specs=_row_spec(fout),
        out_shape=jax.ShapeDtypeStruct((NDP, fout), F32),
    )(x, w, dinv)


def _drug_mid(agg, u, dinv, b, w, fin, fout):
    return pl.pallas_call(
        _drug_mid_body,
        grid=(NDP // ROWB,),
        in_specs=[_row_spec(fin), _row_spec(fin), pl.BlockSpec((ROWB,), lambda i: (i,)),
                  _vec_spec(fin), _full_spec(fin, fout)],
        out_specs=[_row_spec(fout), _row_spec(fin)],
        out_shape=[jax.ShapeDtypeStruct((NDP, fout), F32),
                   jax.ShapeDtypeStruct((NDP, fin), F32)],
    )(agg, u, dinv, b, w)


def _drug_fin(agg, u, dinv, b, fin):
    return pl.pallas_call(
        _drug_fin_body,
        grid=(NDP // ROWB,),
        in_specs=[_row_spec(fin), _row_spec(fin), pl.BlockSpec((ROWB,), lambda i: (i,)),
                  _vec_spec(fin)],
        out_specs=_row_spec(fin),
        out_shape=jax.ShapeDtypeStruct((NDP, fin), F32),
    )(agg, u, dinv, b)


RPT = 208
ACCR = RPT + 8
EBLK = 128
GBLK = 32
CAP = 1664


def _sc_agg_body(nblk, f, u_hbm, sp_hbm, dp_hbm, out_hbm,
                 sblk, dblk, pend_s, pend_l, rows_v, acc, sem):
    w = lax.axis_index("c") * 16 + lax.axis_index("s")
    base = w * RPT
    iota = lax.iota(jnp.int32, 16)

    zvec = jnp.zeros((16,), F32)

    def zacc(m, carry):
        for k in range(16):
            acc[pl.ds(m * 256 + 16 * k, 16)] = zvec
        return carry

    lax.fori_loop(0, ACCR * f // 256, zacc, 0)

    def scan(j, off):
        pltpu.sync_copy(sp_hbm.at[pl.ds(j * EBLK, EBLK)], sblk)
        pltpu.sync_copy(dp_hbm.at[pl.ds(j * EBLK, EBLK)], dblk)
        for k in range(EBLK // 16):
            dv = dblk[pl.ds(16 * k, 16)]
            loc = dv - base
            ok = (loc >= 0) & (loc < RPT)
            cum = plsc.cumsum(ok.astype(jnp.int32))
            idx = off + cum - 1
            plsc.store_scatter(pend_s, [idx], sblk[pl.ds(16 * k, 16)], mask=ok)
            plsc.store_scatter(pend_l, [idx], loc, mask=ok)
            off = off + jnp.sum(ok.astype(jnp.int32))
        return off

    off = lax.fori_loop(0, nblk, scan, jnp.int32(0))

    for m in range(GBLK // 16):
        pad_idx = off + 16 * m + iota
        plsc.store_scatter(pend_s, [pad_idx], jnp.zeros((16,), jnp.int32))
        plsc.store_scatter(pend_l, [pad_idx],
                           jnp.full((16,), RPT, jnp.int32))

    def accum(b, carry):
        pltpu.async_copy(u_hbm.at[pend_s.at[pl.ds(b * GBLK, GBLK)]],
                         rows_v, sem).wait()
        for q in range(GBLK // 16):
            ldvec = pend_l[pl.ds(b * GBLK + 16 * q, 16)]
            for r in range(16):
                ld = ldvec[r]
                for k in range(f // 16):
                    plsc.addupdate(acc.at[pl.ds(ld * f + 16 * k, 16)],
                                   rows_v[16 * q + r, pl.ds(16 * k, 16)])
        return carry

    lax.fori_loop(0, (off + GBLK - 1) // GBLK, accum, 0)

    pltpu.sync_copy(acc.at[pl.ds(0, RPT * f)],
                    out_hbm.at[pl.ds(base * f, RPT * f)])


def _sc_agg(u, sp, dp, f):
    nblk = sp.shape[0] // EBLK
    mesh = plsc.VectorSubcoreMesh(core_axis_name="c", subcore_axis_name="s")
    k = pl.kernel(
        functools.partial(_sc_agg_body, nblk, f),
        out_type=jax.ShapeDtypeStruct((NDP * f,), F32),
        mesh=mesh,
        compiler_params=pltpu.CompilerParams(needs_layout_passes=False),
        scratch_types=[
            pltpu.VMEM((EBLK,), jnp.int32),
            pltpu.VMEM((EBLK,), jnp.int32),
            pltpu.VMEM((CAP,), jnp.int32),
            pltpu.VMEM((CAP,), jnp.int32),
            pltpu.VMEM((GBLK, f), F32),
            pltpu.VMEM((ACCR * f,), F32),
            pltpu.SemaphoreType.DMA,
        ],
    )
    return k(u, sp, dp).reshape(NDP, f)


def _edge_layout(s, d):
    e = s.shape[0]
    tot = -(-e // EBLK) * EBLK
    sp = jnp.zeros((tot,), jnp.int32).at[:e].set(s)
    dp = jnp.full((tot,), NDP, jnp.int32).at[:e].set(d)
    return sp, dp


def _middle_body(pooled_ref, x1_ref, c_ref, wfc_ref, bfc_ref,
                 wg_ref, bg_ref, sel_ref, fc1w_ref, fc1b_ref,
                 w0t_ref, w0b_ref, a_ref, b_ref):
    C = c_ref[...]
    deg = jnp.sum(C, axis=1)
    dinv = jnp.where(deg > 0, jax.lax.rsqrt(deg), 0.0)
    Adj = dinv[:, None] * C * dinv[None, :]

    gfeat = jax.nn.relu(_bdot(pooled_ref[...], wfc_ref[...])
                        + bfc_ref[...][None, :])
    rows = jax.lax.broadcasted_iota(jnp.int32, (512, 1), 0)
    xcat = x1_ref[...] + jnp.where(rows < N_DRUGS, _pad_rows(gfeat, 512), 0.0)

    sel = sel_ref[...][:, None]
    xsel = jnp.zeros((512, 512), F32)
    for l in range(3):
        xl = jax.nn.relu(_bdot(Adj.astype(F32), _bdot(xcat, wg_ref[l]))
                         + bg_ref[l][None, :])
        xsel = xsel + jnp.where(sel == l, xl, 0.0)
    xf = jax.nn.relu(_bdot(xsel, fc1w_ref[...]) + fc1b_ref[...][None, :])
    a_ref[...] = _bdot(xf, w0t_ref[0]) + _bdot(xcat, w0t_ref[1])
    b_ref[...] = _bdot(xf, w0b_ref[0]) + _bdot(xcat, w0b_ref[1])


def _pad_rows(a, n):
    return jnp.pad(a, ((0, n - a.shape[0]), (0, 0)))


def _middle(pooled, x1p, C, wfc, bfc, wg, bg, sel, fc1w, fc1b, w0t, w0b):
    fs = _full_spec
    return pl.pallas_call(
        _middle_body,
        grid=(1,),
        in_specs=[fs(224, 384), fs(512, 512), fs(512, 512), fs(384, 512),
                  _vec_spec(512), pl.BlockSpec((3, 512, 512), lambda i: (0, 0, 0)),
                  pl.BlockSpec((3, 512), lambda i: (0, 0)),
                  pl.BlockSpec((512,), lambda i: (0,)), fs(512, 512),
                  _vec_spec(512), pl.BlockSpec((2, 512, 512), lambda i: (0, 0, 0)),
                  pl.BlockSpec((2, 512, 512), lambda i: (0, 0, 0))],
        out_specs=[fs(512, 512), fs(512, 512)],
        out_shape=[jax.ShapeDtypeStruct((512, 512), F32),
                   jax.ShapeDtypeStruct((512, 512), F32)],
    )(pooled, x1p, C, wfc, bfc, wg, bg, sel, fc1w, fc1b, w0t, w0b)


def _mlp_tail(z0, w1_ref, b1_ref, w2_ref, b2_ref, wl_ref, bl_ref):
    h = jax.nn.relu(z0)
    h = jax.nn.relu(_bdot(h, w1_ref[...]) + b1_ref[...][None, :])
    h = jax.nn.relu(_bdot(h, w2_ref[...]) + b2_ref[...][None, :])
    logit = jnp.sum(h * wl_ref[...][None, :], axis=1) + bl_ref[0]
    return jax.nn.sigmoid(logit)


def _out2_body(a2_ref, b2_ref, b0_ref, w1_ref, b1_ref, w2_ref, b2w_ref,
               wl_ref, bl_ref, o_ref, *, bi):
    z0 = (b2_ref[...][:, None, :] + a2_ref[...][None, :, :]
          + b0_ref[...][None, None, :]).reshape(bi * 272, 512)
    o_ref[...] = _mlp_tail(z0, w1_ref, b1_ref, w2_ref, b2w_ref,
                           wl_ref, bl_ref).reshape(bi, 272)


def _out2(a2, b2, b0, w1, b1, w2, b2w, wl, bl, bi=16):
    nblk = 224 // bi
    return pl.pallas_call(
        functools.partial(_out2_body, bi=bi),
        grid=(nblk,),
        in_specs=[_full_spec(272, 512), pl.BlockSpec((bi, 512), lambda i: (i, 0)),
                  _vec_spec(512), _full_spec(512, 512), _vec_spec(512),
                  _full_spec(512, 512), _vec_spec(512), _vec_spec(512),
                  _vec_spec(8)],
        out_specs=pl.BlockSpec((bi, 272), lambda i: (i, 0)),
        out_shape=jax.ShapeDtypeStruct((224, 272), F32),
    )(a2, b2, b0, w1, b1, w2, b2w, wl, bl)


def _out1_body(z0_ref, w1_ref, b1_ref, w2_ref, b2_ref, wl_ref, bl_ref, o_ref):
    o_ref[...] = _mlp_tail(z0_ref[...], w1_ref, b1_ref, w2_ref, b2_ref,
                           wl_ref, bl_ref)


def _out1(z0, w1, b1, w2, b2, wl, bl):
    return pl.pallas_call(
        _out1_body,
        grid=(8,),
        in_specs=[pl.BlockSpec((1024, 512), lambda i: (i, 0)),
                  _full_spec(512, 512), _vec_spec(512), _full_spec(512, 512),
                  _vec_spec(512), _vec_spec(512), _vec_spec(8)],
        out_specs=pl.BlockSpec((1024,), lambda i: (i,)),
        out_shape=jax.ShapeDtypeStruct((8192,), F32),
    )(z0, w1, b1, w2, b2, wl, bl)


def kernel(x1, edges, hop, edges2, drug_x, drug_edge_index, drug_batch, params):
    p = params
    s, dd = drug_edge_index[0], drug_edge_index[1]

    inv = 1.0 / np.sqrt(1.0 + BN_EPS)
    g0, g1, g2 = p['bn_g0'] * inv, p['bn_g1'] * inv, p['bn_g2'] * inv
    w1p = _pad2(g0[:, None] * p['d_W1'], 512, 512)
    b1p = _pad1(p['bn_b0'] @ p['d_W1'] + p['d_b1'], 512)
    w2p = _pad2(g1[:, None] * p['d_W2'], 512, 512)
    b2p = _pad1(p['bn_b1'] @ p['d_W2'] + p['d_b2'], 512)
    wlp = _pad1((g2[:, None] * p['d_Wl'])[:, 0], 512)
    blp = _pad1(p['bn_b2'] @ p['d_Wl'] + p['d_bl'], 8)
    b0p = _pad1(p['d_b0'], 512)

    gw1 = _pad2(p['g_W1'], 128, 128)
    gw2 = _pad2(p['g_W2'], 128, 256)
    gw3 = _pad2(p['g_W3'], 256, 384)
    gwfc = _pad2(p['g_Wfc'], 384, 512)
    gb1 = _pad1(p['g_b1'], 128)
    gb2 = _pad1(p['g_b2'], 256)
    gb3 = _pad1(p['g_b3'], 384)
    gbfc = _pad1(p['g_bfc'], 512)
    wg = jnp.stack([_pad2(p['W_g%d' % l], 512, 512) for l in range(3)])
    bg = jnp.stack([_pad1(p['b_g%d' % l], 512) for l in range(3)])
    fc1w = _pad2(p['fc1_W'], 512, 512)
    fc1b = _pad1(p['fc1_b'], 512)
    w0t = jnp.stack([_pad2(p['d_W0'][:489], 512, 512),
                     _pad2(p['d_W0'][489:978], 512, 512)])
    w0b = jnp.stack([_pad2(p['d_W0'][978:978 + 489], 512, 512),
                     _pad2(p['d_W0'][978 + 489:], 512, 512)])

    deg = jnp.zeros((ND,), F32).at[dd].add(1.0) + 1.0
    dinv = _pad1(deg ** -0.5, NDP)

    spi, dpi = _edge_layout(s, dd)
    xq = _pad2(drug_x, NDP, 128)
    u1 = _drug_l1(xq, gw1, dinv, 128, 128)
    agg1 = _sc_agg(u1, spi, dpi, 128)
    u2, _ = _drug_mid(agg1, u1, dinv, gb1, gw2, 128, 256)
    agg2 = _sc_agg(u2, spi, dpi, 256)
    u3, _ = _drug_mid(agg2, u2, dinv, gb2, gw3, 256, 384)
    agg3 = _sc_agg(u3, spi, dpi, 384)
    x4 = _drug_fin(agg3, u3, dinv, gb3, 384)

    pooled = jax.ops.segment_max(x4[:ND], drug_batch, num_segments=N_DRUGS)
    pooled = jnp.where(jnp.isfinite(pooled), pooled, 0.0)
    pooled = _pad2(pooled, 224, 384)

    C = (jnp.zeros((512, 512), F32).at[edges[1], edges[0]].add(1.0)
         .at[jnp.arange(N_NODES), jnp.arange(N_NODES)].add(1.0))

    x1p = _pad2(x1, 512, 512)
    sel = _pad1(jnp.where(hop == 0, 2, hop - 1).astype(jnp.int32), 512)
    A, B = _middle(pooled, x1p, C, gwfc, gbfc, wg, bg, sel, fc1w, fc1b,
                   w0t, w0b)

    a2 = _pad_rows(A[N_DRUGS:N_NODES], 272)
    b2 = B[:224]
    out2 = _out2(a2, b2, b0p, w1p, b1p, w2p, b2p, wlp, blp)[:N_DRUGS, :271]

    z0 = A[edges2[1]] + B[edges2[0]] + b0p[None, :]
    out1 = _out1(z0, w1p, b1p, w2p, b2p, wlp, blp)

    return out1, out2


# speedup vs baseline: 1.3068x; 1.3068x over previous
import functools

import jax
import jax.numpy as jnp
import numpy as np
from jax import lax
from jax.experimental import pallas as pl
from jax.experimental.pallas import tpu as pltpu
from jax.experimental.pallas import tpu_sc as plsc

N_DRUGS = 218
N_NODES = 489
BN_EPS = 1e-5
F32 = jnp.float32
BF16 = jnp.bfloat16

ND = 6540
NDP = 6656
ROWB = 512


def _pad2(a, r, c):
    return jnp.zeros((r, c), a.dtype).at[: a.shape[0], : a.shape[1]].set(a)


def _pad1(a, n):
    return jnp.zeros((n,), a.dtype).at[: a.shape[0]].set(a)


def _bdot(a, b):
    return jax.lax.dot(a.astype(BF16), b.astype(BF16),
                       preferred_element_type=F32)


def _drug_l1_body(x_ref, w_ref, dinv_ref, u_ref):
    u_ref[...] = dinv_ref[...][:, None] * _bdot(x_ref[...], w_ref[...])


def _drug_mid_body(agg_ref, u_ref, dinv_ref, b_ref, w_ref, uo_ref, x_ref):
    dinv = dinv_ref[...][:, None]
    x = jax.nn.relu(dinv * (agg_ref[...] + u_ref[...]) + b_ref[...][None, :])
    x_ref[...] = x
    uo_ref[...] = dinv * _bdot(x, w_ref[...])


def _drug_fin_body(agg_ref, u_ref, dinv_ref, b_ref, x_ref):
    dinv = dinv_ref[...][:, None]
    x_ref[...] = jax.nn.relu(dinv * (agg_ref[...] + u_ref[...])
                             + b_ref[...][None, :])


def _row_spec(c):
    return pl.BlockSpec((ROWB, c), lambda i: (i, 0))


def _vec_spec(n):
    return pl.BlockSpec((n,), lambda i: (0,))


def _full_spec(r, c):
    return pl.BlockSpec((r, c), lambda i: (0, 0))


def _drug_l1(x, w, dinv, fin, fout):
    return pl.pallas_call(
        _drug_l1_body,
        grid=(NDP // ROWB,),
        in_specs=[_row_spec(fin), _full_spec(fin, fout), pl.BlockSpec((ROWB,), lambda i: (i,))],
        out_specs=_row_spec(fout),
        out_shape=jax.ShapeDtypeStruct((NDP, fout), F32),
    )(x, w, dinv)


def _drug_mid(agg, u, dinv, b, w, fin, fout):
    return pl.pallas_call(
        _drug_mid_body,
        grid=(NDP // ROWB,),
        in_specs=[_row_spec(fin), _row_spec(fin), pl.BlockSpec((ROWB,), lambda i: (i,)),
                  _vec_spec(fin), _full_spec(fin, fout)],
        out_specs=[_row_spec(fout), _row_spec(fin)],
        out_shape=[jax.ShapeDtypeStruct((NDP, fout), F32),
                   jax.ShapeDtypeStruct((NDP, fin), F32)],
    )(agg, u, dinv, b, w)


def _drug_fin(agg, u, dinv, b, fin):
    return pl.pallas_call(
        _drug_fin_body,
        grid=(NDP // ROWB,),
        in_specs=[_row_spec(fin), _row_spec(fin), pl.BlockSpec((ROWB,), lambda i: (i,)),
                  _vec_spec(fin)],
        out_specs=_row_spec(fin),
        out_shape=jax.ShapeDtypeStruct((NDP, fin), F32),
    )(agg, u, dinv, b)


RPT = 208
ACCR = RPT + 8
EBLK = 1024
GBLK = 32
CAPL = 128
MCAP = 16 * CAPL + 2 * GBLK


def _sc_fire(u_hbm, msrc, b, buf, sem):
    pltpu.async_copy(u_hbm.at[msrc.at[pl.ds(b * GBLK, GBLK)]], buf, sem)


def _sc_wait(u_hbm, buf, sem):
    pltpu.make_async_copy(u_hbm.at[pl.ds(0, GBLK)], buf, sem).wait()


def _sc_agg_body(nblk, f, u_hbm, sp_hbm, dp_hbm, out_hbm,
                 sblk, dblk, pend_s, pend_l, msrc, mloc,
                 rows_a, rows_b, acc, sem_a, sem_b):
    w = lax.axis_index("c") * 16 + lax.axis_index("s")
    base = w * RPT
    iota = lax.iota(jnp.int32, 16)
    lane_base = iota * CAPL

    zvec = jnp.zeros((16,), F32)

    def zacc(m, carry):
        for k in range(16):
            acc[pl.ds(m * 256 + 16 * k, 16)] = zvec
        return carry

    lax.fori_loop(0, ACCR * f // 256, zacc, 0)

    def scan(j, cnt):
        pltpu.sync_copy(sp_hbm.at[pl.ds(j * EBLK, EBLK)], sblk)
        pltpu.sync_copy(dp_hbm.at[pl.ds(j * EBLK, EBLK)], dblk)
        for k in range(EBLK // 16):
            dv = dblk[pl.ds(16 * k, 16)]
            loc = dv - base
            ok = (loc >= 0) & (loc < RPT)
            idx = lane_base + jnp.minimum(cnt, CAPL - 1)
            plsc.store_scatter(pend_s, [idx], sblk[pl.ds(16 * k, 16)], mask=ok)
            plsc.store_scatter(pend_l, [idx], loc, mask=ok)
            cnt = cnt + jnp.where(ok, 1, 0)
        return cnt

    cnt = lax.fori_loop(0, nblk, scan, jnp.zeros((16,), jnp.int32))

    pfx = plsc.cumsum(cnt)
    total = pfx[15]

    for l in range(16):
        nl = cnt[l]
        st = pfx[l] - nl

        def mv(tt, carry, l=l, nl=nl, st=st):
            pos = 16 * tt + iota
            m = pos < nl
            sv = pend_s[pl.ds(l * CAPL + 16 * tt, 16)]
            lv = pend_l[pl.ds(l * CAPL + 16 * tt, 16)]
            plsc.store_scatter(msrc, [st + pos], sv, mask=m)
            plsc.store_scatter(mloc, [st + pos], lv, mask=m)
            return carry

        lax.fori_loop(0, (nl + 15) // 16, mv, 0)

    for m in range(GBLK // 16):
        plsc.store_scatter(msrc, [total + 16 * m + iota],
                           jnp.zeros((16,), jnp.int32))
        plsc.store_scatter(mloc, [total + 16 * m + iota],
                           jnp.full((16,), RPT, jnp.int32))

    nb = total // GBLK + 1

    def accum(b, buf):
        for q in range(GBLK // 16):
            ldvec = mloc[pl.ds(b * GBLK + 16 * q, 16)]
            for r in range(16):
                ld = ldvec[r]
                for k in range(f // 16):
                    plsc.addupdate(acc.at[pl.ds(ld * f + 16 * k, 16)],
                                   buf[16 * q + r, pl.ds(16 * k, 16)])

    _sc_fire(u_hbm, msrc, 0, rows_a, sem_a)

    def pairs(i, carry):
        b0 = 2 * i

        @pl.when(b0 + 1 < nb)
        def _():
            _sc_fire(u_hbm, msrc, b0 + 1, rows_b, sem_b)
        _sc_wait(u_hbm, rows_a, sem_a)
        accum(b0, rows_a)

        @pl.when(b0 + 2 < nb)
        def _():
            _sc_fire(u_hbm, msrc, b0 + 2, rows_a, sem_a)

        @pl.when(b0 + 1 < nb)
        def _():
            _sc_wait(u_hbm, rows_b, sem_b)
            accum(b0 + 1, rows_b)
        return carry

    lax.fori_loop(0, (nb + 1) // 2, pairs, 0)

    pltpu.sync_copy(acc.at[pl.ds(0, RPT * f)],
                    out_hbm.at[pl.ds(base * f, RPT * f)])


def _sc_agg(u, sp, dp, f):
    nblk = sp.shape[0] // EBLK
    mesh = plsc.VectorSubcoreMesh(core_axis_name="c", subcore_axis_name="s")
    k = pl.kernel(
        functools.partial(_sc_agg_body, nblk, f),
        out_type=jax.ShapeDtypeStruct((NDP * f,), F32),
        mesh=mesh,
        compiler_params=pltpu.CompilerParams(needs_layout_passes=False),
        scratch_types=[
            pltpu.VMEM((EBLK,), jnp.int32),
            pltpu.VMEM((EBLK,), jnp.int32),
            pltpu.VMEM((16 * CAPL,), jnp.int32),
            pltpu.VMEM((16 * CAPL,), jnp.int32),
            pltpu.VMEM((MCAP,), jnp.int32),
            pltpu.VMEM((MCAP,), jnp.int32),
            pltpu.VMEM((GBLK, f), F32),
            pltpu.VMEM((GBLK, f), F32),
            pltpu.VMEM((ACCR * f,), F32),
            pltpu.SemaphoreType.DMA,
            pltpu.SemaphoreType.DMA,
        ],
    )
    return k(u, sp, dp).reshape(NDP, f)


def _edge_layout(s, d):
    e = s.shape[0]
    tot = -(-e // EBLK) * EBLK
    sp = jnp.zeros((tot,), jnp.int32).at[:e].set(s)
    dp = jnp.full((tot,), NDP, jnp.int32).at[:e].set(d)
    return sp, dp


def _middle_body(pooled_ref, x1_ref, c_ref, wfc_ref, bfc_ref,
                 wg_ref, bg_ref, sel_ref, fc1w_ref, fc1b_ref,
                 w0t_ref, w0b_ref, a_ref, b_ref):
    C = c_ref[...]
    deg = jnp.sum(C, axis=1)
    dinv = jnp.where(deg > 0, jax.lax.rsqrt(deg), 0.0)
    Adj = dinv[:, None] * C * dinv[None, :]

    gfeat = jax.nn.relu(_bdot(pooled_ref[...], wfc_ref[...])
                        + bfc_ref[...][None, :])
    rows = jax.lax.broadcasted_iota(jnp.int32, (512, 1), 0)
    xcat = x1_ref[...] + jnp.where(rows < N_DRUGS, _pad_rows(gfeat, 512), 0.0)

    sel = sel_ref[...][:, None]
    xsel = jnp.zeros((512, 512), F32)
    for l in range(3):
        xl = jax.nn.relu(_bdot(Adj.astype(F32), _bdot(xcat, wg_ref[l]))
                         + bg_ref[l][None, :])
        xsel = xsel + jnp.where(sel == l, xl, 0.0)
    xf = jax.nn.relu(_bdot(xsel, fc1w_ref[...]) + fc1b_ref[...][None, :])
    a_ref[...] = _bdot(xf, w0t_ref[0]) + _bdot(xcat, w0t_ref[1])
    b_ref[...] = _bdot(xf, w0b_ref[0]) + _bdot(xcat, w0b_ref[1])


def _pad_rows(a, n):
    return jnp.pad(a, ((0, n - a.shape[0]), (0, 0)))


def _middle(pooled, x1p, C, wfc, bfc, wg, bg, sel, fc1w, fc1b, w0t, w0b):
    fs = _full_spec
    return pl.pallas_call(
        _middle_body,
        grid=(1,),
        in_specs=[fs(224, 384), fs(512, 512), fs(512, 512), fs(384, 512),
                  _vec_spec(512), pl.BlockSpec((3, 512, 512), lambda i: (0, 0, 0)),
                  pl.BlockSpec((3, 512), lambda i: (0, 0)),
                  pl.BlockSpec((512,), lambda i: (0,)), fs(512, 512),
                  _vec_spec(512), pl.BlockSpec((2, 512, 512), lambda i: (0, 0, 0)),
                  pl.BlockSpec((2, 512, 512), lambda i: (0, 0, 0))],
        out_specs=[fs(512, 512), fs(512, 512)],
        out_shape=[jax.ShapeDtypeStruct((512, 512), F32),
                   jax.ShapeDtypeStruct((512, 512), F32)],
    )(pooled, x1p, C, wfc, bfc, wg, bg, sel, fc1w, fc1b, w0t, w0b)


def _mlp_tail(z0, w1_ref, b1_ref, w2_ref, b2_ref, wl_ref, bl_ref):
    h = jax.nn.relu(z0)
    h = jax.nn.relu(_bdot(h, w1_ref[...]) + b1_ref[...][None, :])
    h = jax.nn.relu(_bdot(h, w2_ref[...]) + b2_ref[...][None, :])
    logit = jnp.sum(h * wl_ref[...][None, :], axis=1) + bl_ref[0]
    return jax.nn.sigmoid(logit)


def _out2_body(a2_ref, b2_ref, b0_ref, w1_ref, b1_ref, w2_ref, b2w_ref,
               wl_ref, bl_ref, o_ref, *, bi):
    z0 = (b2_ref[...][:, None, :] + a2_ref[...][None, :, :]
          + b0_ref[...][None, None, :]).reshape(bi * 272, 512)
    o_ref[...] = _mlp_tail(z0, w1_ref, b1_ref, w2_ref, b2w_ref,
                           wl_ref, bl_ref).reshape(bi, 272)


def _out2(a2, b2, b0, w1, b1, w2, b2w, wl, bl, bi=16):
    nblk = 224 // bi
    return pl.pallas_call(
        functools.partial(_out2_body, bi=bi),
        grid=(nblk,),
        in_specs=[_full_spec(272, 512), pl.BlockSpec((bi, 512), lambda i: (i, 0)),
                  _vec_spec(512), _full_spec(512, 512), _vec_spec(512),
                  _full_spec(512, 512), _vec_spec(512), _vec_spec(512),
                  _vec_spec(8)],
        out_specs=pl.BlockSpec((bi, 272), lambda i: (i, 0)),
        out_shape=jax.ShapeDtypeStruct((224, 272), F32),
    )(a2, b2, b0, w1, b1, w2, b2w, wl, bl)


def _out1_body(z0_ref, w1_ref, b1_ref, w2_ref, b2_ref, wl_ref, bl_ref, o_ref):
    o_ref[...] = _mlp_tail(z0_ref[...], w1_ref, b1_ref, w2_ref, b2_ref,
                           wl_ref, bl_ref)


def _out1(z0, w1, b1, w2, b2, wl, bl):
    return pl.pallas_call(
        _out1_body,
        grid=(8,),
        in_specs=[pl.BlockSpec((1024, 512), lambda i: (i, 0)),
                  _full_spec(512, 512), _vec_spec(512), _full_spec(512, 512),
                  _vec_spec(512), _vec_spec(512), _vec_spec(8)],
        out_specs=pl.BlockSpec((1024,), lambda i: (i,)),
        out_shape=jax.ShapeDtypeStruct((8192,), F32),
    )(z0, w1, b1, w2, b2, wl, bl)


def kernel(x1, edges, hop, edges2, drug_x, drug_edge_index, drug_batch, params):
    p = params
    s, dd = drug_edge_index[0], drug_edge_index[1]

    inv = 1.0 / np.sqrt(1.0 + BN_EPS)
    g0, g1, g2 = p['bn_g0'] * inv, p['bn_g1'] * inv, p['bn_g2'] * inv
    w1p = _pad2(g0[:, None] * p['d_W1'], 512, 512)
    b1p = _pad1(p['bn_b0'] @ p['d_W1'] + p['d_b1'], 512)
    w2p = _pad2(g1[:, None] * p['d_W2'], 512, 512)
    b2p = _pad1(p['bn_b1'] @ p['d_W2'] + p['d_b2'], 512)
    wlp = _pad1((g2[:, None] * p['d_Wl'])[:, 0], 512)
    blp = _pad1(p['bn_b2'] @ p['d_Wl'] + p['d_bl'], 8)
    b0p = _pad1(p['d_b0'], 512)

    gw1 = _pad2(p['g_W1'], 128, 128)
    gw2 = _pad2(p['g_W2'], 128, 256)
    gw3 = _pad2(p['g_W3'], 256, 384)
    gwfc = _pad2(p['g_Wfc'], 384, 512)
    gb1 = _pad1(p['g_b1'], 128)
    gb2 = _pad1(p['g_b2'], 256)
    gb3 = _pad1(p['g_b3'], 384)
    gbfc = _pad1(p['g_bfc'], 512)
    wg = jnp.stack([_pad2(p['W_g%d' % l], 512, 512) for l in range(3)])
    bg = jnp.stack([_pad1(p['b_g%d' % l], 512) for l in range(3)])
    fc1w = _pad2(p['fc1_W'], 512, 512)
    fc1b = _pad1(p['fc1_b'], 512)
    w0t = jnp.stack([_pad2(p['d_W0'][:489], 512, 512),
                     _pad2(p['d_W0'][489:978], 512, 512)])
    w0b = jnp.stack([_pad2(p['d_W0'][978:978 + 489], 512, 512),
                     _pad2(p['d_W0'][978 + 489:], 512, 512)])

    deg = jnp.zeros((ND,), F32).at[dd].add(1.0) + 1.0
    dinv = _pad1(deg ** -0.5, NDP)

    spi, dpi = _edge_layout(s, dd)
    xq = _pad2(drug_x, NDP, 128)
    u1 = _drug_l1(xq, gw1, dinv, 128, 128)
    agg1 = _sc_agg(u1, spi, dpi, 128)
    u2, _ = _drug_mid(agg1, u1, dinv, gb1, gw2, 128, 256)
    agg2 = _sc_agg(u2, spi, dpi, 256)
    u3, _ = _drug_mid(agg2, u2, dinv, gb2, gw3, 256, 384)
    agg3 = _sc_agg(u3, spi, dpi, 384)
    x4 = _drug_fin(agg3, u3, dinv, gb3, 384)

    pooled = jax.ops.segment_max(x4[:ND], drug_batch, num_segments=N_DRUGS)
    pooled = jnp.where(jnp.isfinite(pooled), pooled, 0.0)
    pooled = _pad2(pooled, 224, 384)

    C = (jnp.zeros((512, 512), F32).at[edges[1], edges[0]].add(1.0)
         .at[jnp.arange(N_NODES), jnp.arange(N_NODES)].add(1.0))

    x1p = _pad2(x1, 512, 512)
    sel = _pad1(jnp.where(hop == 0, 2, hop - 1).astype(jnp.int32), 512)
    A, B = _middle(pooled, x1p, C, gwfc, gbfc, wg, bg, sel, fc1w, fc1b,
                   w0t, w0b)

    a2 = _pad_rows(A[N_DRUGS:N_NODES], 272)
    b2 = B[:224]
    out2 = _out2(a2, b2, b0p, w1p, b1p, w2p, b2p, wlp, blp)[:N_DRUGS, :271]

    z0 = A[edges2[1]] + B[edges2[0]] + b0p[None, :]
    out1 = _out1(z0, w1p, b1p, w2p, b2p, wlp, blp)

    return out1, out2


# speedup vs baseline: 1.7870x; 1.3675x over previous
import functools

import jax
import jax.numpy as jnp
import numpy as np
from jax import lax
from jax.experimental import pallas as pl
from jax.experimental.pallas import tpu as pltpu
from jax.experimental.pallas import tpu_sc as plsc

N_DRUGS = 218
N_NODES = 489
BN_EPS = 1e-5
F32 = jnp.float32
BF16 = jnp.bfloat16

ND = 6540
NDP = 6656
ROWB = 512


def _pad2(a, r, c):
    return jnp.zeros((r, c), a.dtype).at[: a.shape[0], : a.shape[1]].set(a)


def _pad1(a, n):
    return jnp.zeros((n,), a.dtype).at[: a.shape[0]].set(a)


def _bdot(a, b):
    return jax.lax.dot(a.astype(BF16), b.astype(BF16),
                       preferred_element_type=F32)


def _drug_l1_body(x_ref, w_ref, dinv_ref, u_ref):
    u_ref[...] = dinv_ref[...][:, None] * _bdot(x_ref[...], w_ref[...])


def _drug_mid_body(agg_ref, u_ref, dinv_ref, b_ref, w_ref, uo_ref, x_ref):
    dinv = dinv_ref[...][:, None]
    x = jax.nn.relu(dinv * (agg_ref[...] + u_ref[...]) + b_ref[...][None, :])
    x_ref[...] = x
    uo_ref[...] = dinv * _bdot(x, w_ref[...])


def _drug_fin_body(agg_ref, u_ref, dinv_ref, b_ref, x_ref):
    dinv = dinv_ref[...][:, None]
    x_ref[...] = jax.nn.relu(dinv * (agg_ref[...] + u_ref[...])
                             + b_ref[...][None, :])


def _row_spec(c):
    return pl.BlockSpec((ROWB, c), lambda i: (i, 0))


def _vec_spec(n):
    return pl.BlockSpec((n,), lambda i: (0,))


def _full_spec(r, c):
    return pl.BlockSpec((r, c), lambda i: (0, 0))


def _drug_l1(x, w, dinv, fin, fout):
    return pl.pallas_call(
        _drug_l1_body,
        grid=(NDP // ROWB,),
        in_specs=[_row_spec(fin), _full_spec(fin, fout), pl.BlockSpec((ROWB,), lambda i: (i,))],
        out_specs=_row_spec(fout),
        out_shape=jax.ShapeDtypeStruct((NDP, fout), F32),
    )(x, w, dinv)


def _drug_mid(agg, u, dinv, b, w, fin, fout):
    return pl.pallas_call(
        _drug_mid_body,
        grid=(NDP // ROWB,),
        in_specs=[_row_spec(fin), _row_spec(fin), pl.BlockSpec((ROWB,), lambda i: (i,)),
                  _vec_spec(fin), _full_spec(fin, fout)],
        out_specs=[_row_spec(fout), _row_spec(fin)],
        out_shape=[jax.ShapeDtypeStruct((NDP, fout), F32),
                   jax.ShapeDtypeStruct((NDP, fin), F32)],
    )(agg, u, dinv, b, w)


def _drug_fin(agg, u, dinv, b, fin):
    return pl.pallas_call(
        _drug_fin_body,
        grid=(NDP // ROWB,),
        in_specs=[_row_spec(fin), _row_spec(fin), pl.BlockSpec((ROWB,), lambda i: (i,)),
                  _vec_spec(fin)],
        out_specs=_row_spec(fin),
        out_shape=jax.ShapeDtypeStruct((NDP, fin), F32),
    )(agg, u, dinv, b)


RPT = 208
ACCR = RPT + 8
EBLK = 1024
GBLK = 32
CAPL = 128
MCAP = 16 * CAPL + 2 * GBLK


def _sc_fire(u_hbm, msrc, b, buf, sem):
    pltpu.async_copy(u_hbm.at[msrc.at[pl.ds(b * GBLK, GBLK)]], buf, sem)


def _sc_wait(u_hbm, buf, sem):
    pltpu.make_async_copy(u_hbm.at[pl.ds(0, GBLK)], buf, sem).wait()


def _sc_agg_body(nblk, f, u_hbm, sp_hbm, dp_hbm, out_hbm,
                 sblk, dblk, pend_s, pend_l, msrc, mloc,
                 rows_a, rows_b, acc, sem_a, sem_b):
    w = lax.axis_index("c") * 16 + lax.axis_index("s")
    base = w * RPT
    iota = lax.iota(jnp.int32, 16)
    lane_base = iota * CAPL

    zvec = jnp.zeros((16,), F32)

    def zacc(m, carry):
        for k in range(16):
            acc[pl.ds(m * 256 + 16 * k, 16)] = zvec
        return carry

    lax.fori_loop(0, ACCR * f // 256, zacc, 0)

    def scan(j, cnt):
        pltpu.sync_copy(sp_hbm.at[pl.ds(j * EBLK, EBLK)], sblk)
        pltpu.sync_copy(dp_hbm.at[pl.ds(j * EBLK, EBLK)], dblk)
        for k in range(EBLK // 16):
            dv = dblk[pl.ds(16 * k, 16)]
            loc = dv - base
            ok = (loc >= 0) & (loc < RPT)
            idx = lane_base + jnp.minimum(cnt, CAPL - 1)
            plsc.store_scatter(pend_s, [idx], sblk[pl.ds(16 * k, 16)], mask=ok)
            plsc.store_scatter(pend_l, [idx], loc, mask=ok)
            cnt = cnt + jnp.where(ok, 1, 0)
        return cnt

    cnt = lax.fori_loop(0, nblk, scan, jnp.zeros((16,), jnp.int32))

    pfx = plsc.cumsum(cnt)
    total = pfx[15]

    for l in range(16):
        nl = cnt[l]
        st = pfx[l] - nl

        def mv(tt, carry, l=l, nl=nl, st=st):
            pos = 16 * tt + iota
            m = pos < nl
            sv = pend_s[pl.ds(l * CAPL + 16 * tt, 16)]
            lv = pend_l[pl.ds(l * CAPL + 16 * tt, 16)]
            plsc.store_scatter(msrc, [st + pos], sv, mask=m)
            plsc.store_scatter(mloc, [st + pos], lv, mask=m)
            return carry

        lax.fori_loop(0, (nl + 15) // 16, mv, 0)

    for m in range(GBLK // 16):
        plsc.store_scatter(msrc, [total + 16 * m + iota],
                           jnp.zeros((16,), jnp.int32))
        plsc.store_scatter(mloc, [total + 16 * m + iota],
                           jnp.full((16,), RPT, jnp.int32))

    nb = total // GBLK + 1

    def accum(b, buf):
        for q in range(GBLK // 16):
            ldvec = mloc[pl.ds(b * GBLK + 16 * q, 16)]
            for r in range(16):
                ld = ldvec[r]
                for k in range(f // 16):
                    plsc.addupdate(acc.at[pl.ds(ld * f + 16 * k, 16)],
                                   buf[16 * q + r, pl.ds(16 * k, 16)])

    _sc_fire(u_hbm, msrc, 0, rows_a, sem_a)

    def pairs(i, carry):
        b0 = 2 * i

        @pl.when(b0 + 1 < nb)
        def _():
            _sc_fire(u_hbm, msrc, b0 + 1, rows_b, sem_b)
        _sc_wait(u_hbm, rows_a, sem_a)
        accum(b0, rows_a)

        @pl.when(b0 + 2 < nb)
        def _():
            _sc_fire(u_hbm, msrc, b0 + 2, rows_a, sem_a)

        @pl.when(b0 + 1 < nb)
        def _():
            _sc_wait(u_hbm, rows_b, sem_b)
            accum(b0 + 1, rows_b)
        return carry

    lax.fori_loop(0, (nb + 1) // 2, pairs, 0)

    pltpu.sync_copy(acc.at[pl.ds(0, RPT * f)],
                    out_hbm.at[pl.ds(base * f, RPT * f)])


def _sc_agg(u, sp, dp, f):
    nblk = sp.shape[0] // EBLK
    mesh = plsc.VectorSubcoreMesh(core_axis_name="c", subcore_axis_name="s")
    k = pl.kernel(
        functools.partial(_sc_agg_body, nblk, f),
        out_type=jax.ShapeDtypeStruct((NDP * f,), F32),
        mesh=mesh,
        compiler_params=pltpu.CompilerParams(needs_layout_passes=False),
        scratch_types=[
            pltpu.VMEM((EBLK,), jnp.int32),
            pltpu.VMEM((EBLK,), jnp.int32),
            pltpu.VMEM((16 * CAPL,), jnp.int32),
            pltpu.VMEM((16 * CAPL,), jnp.int32),
            pltpu.VMEM((MCAP,), jnp.int32),
            pltpu.VMEM((MCAP,), jnp.int32),
            pltpu.VMEM((GBLK, f), F32),
            pltpu.VMEM((GBLK, f), F32),
            pltpu.VMEM((ACCR * f,), F32),
            pltpu.SemaphoreType.DMA,
            pltpu.SemaphoreType.DMA,
        ],
    )
    return k(u, sp, dp).reshape(NDP, f)


def _edge_layout(s, d):
    e = s.shape[0]
    tot = -(-e // EBLK) * EBLK
    sp = jnp.zeros((tot,), jnp.int32).at[:e].set(s)
    dp = jnp.full((tot,), NDP, jnp.int32).at[:e].set(d)
    return sp, dp


SEGR = 224
SEGF = 384


def _sc_segmax_body(x_hbm, b_hbm, out_hbm, bb, rbuf, acc, sem):
    w = lax.axis_index("c") * 16 + lax.axis_index("s")
    ninf = jnp.full((16,), -jnp.inf, F32)

    def zacc(m, carry):
        for k in range(16):
            acc[pl.ds(m * 256 + 16 * k, 16)] = ninf
        return carry

    lax.fori_loop(0, (SEGR * SEGF + 256) // 256, zacc, 0)

    pltpu.sync_copy(b_hbm.at[pl.ds(w * RPT, RPT)], bb)

    def chunk(m, carry):
        pltpu.sync_copy(x_hbm.at[pl.ds(w * RPT + 16 * m, 16)], rbuf)
        bv = bb[pl.ds(16 * m, 16)]
        for r in range(16):
            sb = bv[r] * SEGF
            for k in range(SEGF // 16):
                cur = acc[pl.ds(sb + 16 * k, 16)]
                acc[pl.ds(sb + 16 * k, 16)] = jnp.maximum(
                    cur, rbuf[r, pl.ds(16 * k, 16)])
        return carry

    lax.fori_loop(0, RPT // 16, chunk, 0)

    pltpu.sync_copy(acc.at[pl.ds(0, SEGR * SEGF)],
                    out_hbm.at[pl.ds(w * SEGR * SEGF, SEGR * SEGF)])


def _sc_segmax(x4, batch_pad):
    mesh = plsc.VectorSubcoreMesh(core_axis_name="c", subcore_axis_name="s")
    k = pl.kernel(
        _sc_segmax_body,
        out_type=jax.ShapeDtypeStruct((32 * SEGR * SEGF,), F32),
        mesh=mesh,
        compiler_params=pltpu.CompilerParams(needs_layout_passes=False),
        scratch_types=[
            pltpu.VMEM((RPT,), jnp.int32),
            pltpu.VMEM((16, SEGF), F32),
            pltpu.VMEM((SEGR * SEGF + 256,), F32),
            pltpu.SemaphoreType.DMA,
        ],
    )
    return k(x4, batch_pad).reshape(32, SEGR, SEGF)


CROWS = 16


def _sc_counts_scan(idx_hbm, nblk, stage, pend, lane_base, to_local):
    def scan(j, cnt):
        pltpu.sync_copy(idx_hbm.at[pl.ds(j * EBLK, EBLK)], stage)
        for k in range(EBLK // 16):
            val, ok = to_local(stage[pl.ds(16 * k, 16)], k)
            idx = lane_base + jnp.minimum(cnt, CAPL - 1)
            plsc.store_scatter(pend, [idx], val, mask=ok)
            cnt = cnt + jnp.where(ok, 1, 0)
        return cnt

    return lax.fori_loop(0, nblk, scan, jnp.zeros((16,), jnp.int32))


def _sc_counts_merge(pend, merged, cnt, dump, iota):
    pfx = plsc.cumsum(cnt)
    total = pfx[15]
    for l in range(16):
        nl = cnt[l]
        st = pfx[l] - nl

        def mv(tt, carry, l=l, nl=nl, st=st):
            pos = 16 * tt + iota
            m = pos < nl
            lv = pend[pl.ds(l * CAPL + 16 * tt, 16)]
            plsc.store_scatter(merged, [st + pos], lv, mask=m)
            return carry

        lax.fori_loop(0, (nl + 15) // 16, mv, 0)
    plsc.store_scatter(merged, [total + iota],
                       jnp.full((16,), dump, jnp.int32))
    return total


def _sc_counts_apply(merged, total, acc, e0):
    def app(t, carry):
        ldvec = merged[pl.ds(16 * t, 16)]
        for r in range(16):
            plsc.addupdate(acc.at[pl.ds(ldvec[r], 16)], e0)
        return carry

    lax.fori_loop(0, total // 16 + 1, app, 0)


def _sc_counts_body(ndblk, neblk, dd_hbm, es_hbm, ed_hbm, deg_hbm, c_hbm,
                    stage, stage2, pend, merged, dacc, cacc, sem):
    w = lax.axis_index("c") * 16 + lax.axis_index("s")
    iota = lax.iota(jnp.int32, 16)
    lane_base = iota * CAPL
    zvec = jnp.zeros((16,), F32)
    e0 = jnp.where(iota == 0, 1.0, 0.0).astype(F32)

    for k in range(RPT // 16 + 1):
        dacc[pl.ds(16 * k, 16)] = zvec

    def zc(m, carry):
        for k in range(16):
            cacc[pl.ds(m * 256 + 16 * k, 16)] = zvec
        return carry

    lax.fori_loop(0, (CROWS * 512 + 256) // 256, zc, 0)

    base = w * RPT

    def loc_deg(dv, k):
        loc = dv - base
        return loc, (loc >= 0) & (loc < RPT)

    cnt = _sc_counts_scan(dd_hbm, ndblk, stage, pend, lane_base, loc_deg)
    total = _sc_counts_merge(pend, merged, cnt, RPT, iota)
    _sc_counts_apply(merged, total, dacc, e0)
    pltpu.sync_copy(dacc.at[pl.ds(0, RPT)], deg_hbm.at[pl.ds(base, RPT)])

    cbase = w * CROWS

    def loc_c(dv, k):
        loc = dv - cbase
        ok = (loc >= 0) & (loc < CROWS)
        sv = stage2[pl.ds(16 * k, 16)]
        return loc * 512 + sv, ok

    def scan2(j, cnt2):
        pltpu.sync_copy(ed_hbm.at[pl.ds(j * EBLK, EBLK)], stage)
        pltpu.sync_copy(es_hbm.at[pl.ds(j * EBLK, EBLK)], stage2)
        for k in range(EBLK // 16):
            val, ok = loc_c(stage[pl.ds(16 * k, 16)], k)
            idx = lane_base + jnp.minimum(cnt2, CAPL - 1)
            plsc.store_scatter(pend, [idx], val, mask=ok)
            cnt2 = cnt2 + jnp.where(ok, 1, 0)
        return cnt2

    cnt2 = lax.fori_loop(0, neblk, scan2, jnp.zeros((16,), jnp.int32))
    total2 = _sc_counts_merge(pend, merged, cnt2, CROWS * 512, iota)
    _sc_counts_apply(merged, total2, cacc, e0)
    pltpu.sync_copy(cacc.at[pl.ds(0, CROWS * 512)],
                    c_hbm.at[pl.ds(cbase * 512, CROWS * 512)])


def _sc_counts(dd, es, ed):
    ndblk = dd.shape[0] // EBLK
    neblk = es.shape[0] // EBLK
    mesh = plsc.VectorSubcoreMesh(core_axis_name="c", subcore_axis_name="s")
    k = pl.kernel(
        functools.partial(_sc_counts_body, ndblk, neblk),
        out_type=[jax.ShapeDtypeStruct((NDP,), F32),
                  jax.ShapeDtypeStruct((512 * 512,), F32)],
        mesh=mesh,
        compiler_params=pltpu.CompilerParams(needs_layout_passes=False),
        scratch_types=[
            pltpu.VMEM((EBLK,), jnp.int32),
            pltpu.VMEM((EBLK,), jnp.int32),
            pltpu.VMEM((16 * CAPL,), jnp.int32),
            pltpu.VMEM((16 * CAPL + 2 * GBLK,), jnp.int32),
            pltpu.VMEM((RPT + 32,), F32),
            pltpu.VMEM((CROWS * 512 + 256 + 32,), F32),
            pltpu.SemaphoreType.DMA,
        ],
    )
    deg, cf = k(dd, es, ed)
    return deg, cf.reshape(512, 512)


def _middle_body(pool_ref, x1_ref, c_ref, wfc_ref, bfc_ref,
                 wg_ref, bg_ref, sel_ref, fc1w_ref, fc1b_ref,
                 w0t_ref, w0b_ref, a_ref, b_ref):
    rr = jax.lax.broadcasted_iota(jnp.int32, (512, 512), 0)
    cc = jax.lax.broadcasted_iota(jnp.int32, (512, 512), 1)
    C = c_ref[...] + jnp.where((rr == cc) & (rr < N_NODES), 1.0, 0.0)
    deg = jnp.sum(C, axis=1)
    dinv = jnp.where(deg > 0, jax.lax.rsqrt(deg), 0.0)
    Adj = dinv[:, None] * C * dinv[None, :]

    pmax = jnp.max(pool_ref[...], axis=0)
    pooled = jnp.where(jnp.isfinite(pmax), pmax, 0.0)
    gfeat = jax.nn.relu(_bdot(pooled, wfc_ref[...])
                        + bfc_ref[...][None, :])
    rows = jax.lax.broadcasted_iota(jnp.int32, (512, 1), 0)
    xcat = x1_ref[...] + jnp.where(rows < N_DRUGS, _pad_rows(gfeat, 512), 0.0)

    sel = sel_ref[...][:, None]
    xsel = jnp.zeros((512, 512), F32)
    for l in range(3):
        xl = jax.nn.relu(_bdot(Adj.astype(F32), _bdot(xcat, wg_ref[l]))
                         + bg_ref[l][None, :])
        xsel = xsel + jnp.where(sel == l, xl, 0.0)
    xf = jax.nn.relu(_bdot(xsel, fc1w_ref[...]) + fc1b_ref[...][None, :])
    a_ref[...] = _bdot(xf, w0t_ref[0]) + _bdot(xcat, w0t_ref[1])
    b_ref[...] = _bdot(xf, w0b_ref[0]) + _bdot(xcat, w0b_ref[1])


def _pad_rows(a, n):
    return jnp.pad(a, ((0, n - a.shape[0]), (0, 0)))


def _middle(pool, x1p, C, wfc, bfc, wg, bg, sel, fc1w, fc1b, w0t, w0b):
    fs = _full_spec
    return pl.pallas_call(
        _middle_body,
        grid=(1,),
        in_specs=[pl.BlockSpec((32, 224, 384), lambda i: (0, 0, 0)),
                  fs(512, 512), fs(512, 512), fs(384, 512),
                  _vec_spec(512), pl.BlockSpec((3, 512, 512), lambda i: (0, 0, 0)),
                  pl.BlockSpec((3, 512), lambda i: (0, 0)),
                  pl.BlockSpec((512,), lambda i: (0,)), fs(512, 512),
                  _vec_spec(512), pl.BlockSpec((2, 512, 512), lambda i: (0, 0, 0)),
                  pl.BlockSpec((2, 512, 512), lambda i: (0, 0, 0))],
        out_specs=[fs(512, 512), fs(512, 512)],
        out_shape=[jax.ShapeDtypeStruct((512, 512), F32),
                   jax.ShapeDtypeStruct((512, 512), F32)],
    )(pool, x1p, C, wfc, bfc, wg, bg, sel, fc1w, fc1b, w0t, w0b)


def _mlp_tail(z0, w1_ref, b1_ref, w2_ref, b2_ref, wl_ref, bl_ref):
    h = jax.nn.relu(z0)
    h = jax.nn.relu(_bdot(h, w1_ref[...]) + b1_ref[...][None, :])
    h = jax.nn.relu(_bdot(h, w2_ref[...]) + b2_ref[...][None, :])
    logit = jnp.sum(h * wl_ref[...][None, :], axis=1) + bl_ref[0]
    return jax.nn.sigmoid(logit)


def _out2_body(a2_ref, b2_ref, b0_ref, w1_ref, b1_ref, w2_ref, b2w_ref,
               wl_ref, bl_ref, o_ref, *, bi):
    z0 = (b2_ref[...][:, None, :] + a2_ref[...][None, :, :]
          + b0_ref[...][None, None, :]).reshape(bi * 272, 512)
    o_ref[...] = _mlp_tail(z0, w1_ref, b1_ref, w2_ref, b2w_ref,
                           wl_ref, bl_ref).reshape(bi, 272)


def _out2(a2, b2, b0, w1, b1, w2, b2w, wl, bl, bi=16):
    nblk = 224 // bi
    return pl.pallas_call(
        functools.partial(_out2_body, bi=bi),
        grid=(nblk,),
        in_specs=[_full_spec(272, 512), pl.BlockSpec((bi, 512), lambda i: (i, 0)),
                  _vec_spec(512), _full_spec(512, 512), _vec_spec(512),
                  _full_spec(512, 512), _vec_spec(512), _vec_spec(512),
                  _vec_spec(8)],
        out_specs=pl.BlockSpec((bi, 272), lambda i: (i, 0)),
        out_shape=jax.ShapeDtypeStruct((224, 272), F32),
    )(a2, b2, b0, w1, b1, w2, b2w, wl, bl)


def _out1_body(rr_ref, dd_ref, a_ref, b_ref, b0_ref,
               w1_ref, b1_ref, w2_ref, b2_ref, wl_ref, bl_ref, o_ref):
    cols = jax.lax.broadcasted_iota(jnp.int32, (1024, 512), 1)
    ohr = (rr_ref[...][:, None] == cols).astype(F32)
    ohd = (dd_ref[...][:, None] == cols).astype(F32)
    z0 = (jax.lax.dot(ohr, a_ref[...], preferred_element_type=F32)
          + jax.lax.dot(ohd, b_ref[...], preferred_element_type=F32)
          + b0_ref[...][None, :])
    o_ref[...] = _mlp_tail(z0, w1_ref, b1_ref, w2_ref, b2_ref,
                           wl_ref, bl_ref)


def _out1(rr, ddx, a, b, b0, w1, b1, w2, b2, wl, bl):
    return pl.pallas_call(
        _out1_body,
        grid=(8,),
        in_specs=[pl.BlockSpec((1024,), lambda i: (i,)),
                  pl.BlockSpec((1024,), lambda i: (i,)),
                  _full_spec(512, 512), _full_spec(512, 512), _vec_spec(512),
                  _full_spec(512, 512), _vec_spec(512), _full_spec(512, 512),
                  _vec_spec(512), _vec_spec(512), _vec_spec(8)],
        out_specs=pl.BlockSpec((1024,), lambda i: (i,)),
        out_shape=jax.ShapeDtypeStruct((8192,), F32),
    )(rr, ddx, a, b, b0, w1, b1, w2, b2, wl, bl)


def kernel(x1, edges, hop, edges2, drug_x, drug_edge_index, drug_batch, params):
    p = params
    s, dd = drug_edge_index[0], drug_edge_index[1]

    inv = 1.0 / np.sqrt(1.0 + BN_EPS)
    g0, g1, g2 = p['bn_g0'] * inv, p['bn_g1'] * inv, p['bn_g2'] * inv
    w1p = _pad2(g0[:, None] * p['d_W1'], 512, 512)
    b1p = _pad1(p['bn_b0'] @ p['d_W1'] + p['d_b1'], 512)
    w2p = _pad2(g1[:, None] * p['d_W2'], 512, 512)
    b2p = _pad1(p['bn_b1'] @ p['d_W2'] + p['d_b2'], 512)
    wlp = _pad1((g2[:, None] * p['d_Wl'])[:, 0], 512)
    blp = _pad1(p['bn_b2'] @ p['d_Wl'] + p['d_bl'], 8)
    b0p = _pad1(p['d_b0'], 512)

    gw1 = _pad2(p['g_W1'], 128, 128)
    gw2 = _pad2(p['g_W2'], 128, 256)
    gw3 = _pad2(p['g_W3'], 256, 384)
    gwfc = _pad2(p['g_Wfc'], 384, 512)
    gb1 = _pad1(p['g_b1'], 128)
    gb2 = _pad1(p['g_b2'], 256)
    gb3 = _pad1(p['g_b3'], 384)
    gbfc = _pad1(p['g_bfc'], 512)
    wg = jnp.stack([_pad2(p['W_g%d' % l], 512, 512) for l in range(3)])
    bg = jnp.stack([_pad1(p['b_g%d' % l], 512) for l in range(3)])
    fc1w = _pad2(p['fc1_W'], 512, 512)
    fc1b = _pad1(p['fc1_b'], 512)
    w0t = jnp.stack([_pad2(p['d_W0'][:489], 512, 512),
                     _pad2(p['d_W0'][489:978], 512, 512)])
    w0b = jnp.stack([_pad2(p['d_W0'][978:978 + 489], 512, 512),
                     _pad2(p['d_W0'][978 + 489:], 512, 512)])

    spi, dpi = _edge_layout(s, dd)
    esp, edp = _edge_layout(edges[0], edges[1])
    deg, C = _sc_counts(dpi, esp, edp)
    dinv = jnp.where(jnp.arange(NDP) < ND, (deg + 1.0) ** -0.5, 0.0)

    xq = _pad2(drug_x, NDP, 128)
    u1 = _drug_l1(xq, gw1, dinv, 128, 128)
    agg1 = _sc_agg(u1, spi, dpi, 128)
    u2, _ = _drug_mid(agg1, u1, dinv, gb1, gw2, 128, 256)
    agg2 = _sc_agg(u2, spi, dpi, 256)
    u3, _ = _drug_mid(agg2, u2, dinv, gb2, gw3, 256, 384)
    agg3 = _sc_agg(u3, spi, dpi, 384)
    x4 = _drug_fin(agg3, u3, dinv, gb3, 384)

    batch_pad = jnp.full((NDP,), N_DRUGS, jnp.int32).at[:ND].set(drug_batch)
    pool = _sc_segmax(x4, batch_pad)

    x1p = _pad2(x1, 512, 512)
    sel = _pad1(jnp.where(hop == 0, 2, hop - 1).astype(jnp.int32), 512)
    A, B = _middle(pool, x1p, C, gwfc, gbfc, wg, bg, sel, fc1w, fc1b,
                   w0t, w0b)

    a2 = _pad_rows(A[N_DRUGS:N_NODES], 272)
    b2 = B[:224]
    out2 = _out2(a2, b2, b0p, w1p, b1p, w2p, b2p, wlp, blp)[:N_DRUGS, :271]

    out1 = _out1(edges2[1], edges2[0], A, B, b0p,
                 w1p, b1p, w2p, b2p, wlp, blp)

    return out1, out2


# speedup vs baseline: 1.9879x; 1.1124x over previous
import functools

import jax
import jax.numpy as jnp
import numpy as np
from jax import lax
from jax.experimental import pallas as pl
from jax.experimental.pallas import tpu as pltpu
from jax.experimental.pallas import tpu_sc as plsc

N_DRUGS = 218
N_NODES = 489
BN_EPS = 1e-5
F32 = jnp.float32
BF16 = jnp.bfloat16

ND = 6540
NDP = 6656
ROWB = 512


def _pad2(a, r, c):
    return jnp.zeros((r, c), a.dtype).at[: a.shape[0], : a.shape[1]].set(a)


def _pad1(a, n):
    return jnp.zeros((n,), a.dtype).at[: a.shape[0]].set(a)


def _bdot(a, b):
    return jax.lax.dot(a.astype(BF16), b.astype(BF16),
                       preferred_element_type=F32)


def _drug_scale_body(x_ref, dinv_ref, v_ref):
    v_ref[...] = dinv_ref[...][:, None] * x_ref[...]


def _drug_mid_body(y_ref, v_ref, dinv_ref, b_ref, w_ref, vo_ref, x_ref):
    dinv = dinv_ref[...][:, None]
    x = jax.nn.relu(dinv * _bdot(y_ref[...] + v_ref[...], w_ref[...])
                    + b_ref[...][None, :])
    x_ref[...] = x
    vo_ref[...] = dinv * x


def _drug_fin_body(y_ref, v_ref, dinv_ref, b_ref, w_ref, x_ref):
    dinv = dinv_ref[...][:, None]
    x_ref[...] = jax.nn.relu(dinv * _bdot(y_ref[...] + v_ref[...], w_ref[...])
                             + b_ref[...][None, :])


def _row_spec(c):
    return pl.BlockSpec((ROWB, c), lambda i: (i, 0))


def _vec_spec(n):
    return pl.BlockSpec((n,), lambda i: (0,))


def _full_spec(r, c):
    return pl.BlockSpec((r, c), lambda i: (0, 0))


def _drug_scale(x, dinv, fin):
    return pl.pallas_call(
        _drug_scale_body,
        grid=(NDP // ROWB,),
        in_specs=[_row_spec(fin), pl.BlockSpec((ROWB,), lambda i: (i,))],
        out_specs=_row_spec(fin),
        out_shape=jax.ShapeDtypeStruct((NDP, fin), F32),
    )(x, dinv)


def _drug_mid(y, v, dinv, b, w, fin, fout):
    return pl.pallas_call(
        _drug_mid_body,
        grid=(NDP // ROWB,),
        in_specs=[_row_spec(fin), _row_spec(fin), pl.BlockSpec((ROWB,), lambda i: (i,)),
                  _vec_spec(fout), _full_spec(fin, fout)],
        out_specs=[_row_spec(fout), _row_spec(fout)],
        out_shape=[jax.ShapeDtypeStruct((NDP, fout), F32),
                   jax.ShapeDtypeStruct((NDP, fout), F32)],
    )(y, v, dinv, b, w)


def _drug_fin(y, v, dinv, b, w, fin, fout):
    return pl.pallas_call(
        _drug_fin_body,
        grid=(NDP // ROWB,),
        in_specs=[_row_spec(fin), _row_spec(fin), pl.BlockSpec((ROWB,), lambda i: (i,)),
                  _vec_spec(fout), _full_spec(fin, fout)],
        out_specs=_row_spec(fout),
        out_shape=jax.ShapeDtypeStruct((NDP, fout), F32),
    )(y, v, dinv, b, w)


RPT = 208
ACCR = RPT + 8
EBLK = 1024
GBLK = 32
CAPL = 128
MCAP = 16 * CAPL + 2 * GBLK


def _sc_fire(u_hbm, msrc, b, buf, sem):
    pltpu.async_copy(u_hbm.at[msrc.at[pl.ds(b * GBLK, GBLK)]], buf, sem)


def _sc_wait(u_hbm, buf, sem):
    pltpu.make_async_copy(u_hbm.at[pl.ds(0, GBLK)], buf, sem).wait()


def _sc_agg_body(nblk, f, u_hbm, sp_hbm, dp_hbm, out_hbm,
                 sblk, dblk, pend_s, pend_l, msrc, mloc,
                 rows_a, rows_b, acc, sem_a, sem_b):
    w = lax.axis_index("c") * 16 + lax.axis_index("s")
    base = w * RPT
    iota = lax.iota(jnp.int32, 16)
    lane_base = iota * CAPL

    zvec = jnp.zeros((16,), F32)

    def zacc(m, carry):
        for k in range(16):
            acc[pl.ds(m * 256 + 16 * k, 16)] = zvec
        return carry

    lax.fori_loop(0, ACCR * f // 256, zacc, 0)

    def scan(j, cnt):
        pltpu.sync_copy(sp_hbm.at[pl.ds(j * EBLK, EBLK)], sblk)
        pltpu.sync_copy(dp_hbm.at[pl.ds(j * EBLK, EBLK)], dblk)
        for k in range(EBLK // 16):
            dv = dblk[pl.ds(16 * k, 16)]
            loc = dv - base
            ok = (loc >= 0) & (loc < RPT)
            idx = lane_base + jnp.minimum(cnt, CAPL - 1)
            plsc.store_scatter(pend_s, [idx], sblk[pl.ds(16 * k, 16)], mask=ok)
            plsc.store_scatter(pend_l, [idx], loc, mask=ok)
            cnt = cnt + jnp.where(ok, 1, 0)
        return cnt

    cnt = lax.fori_loop(0, nblk, scan, jnp.zeros((16,), jnp.int32))

    pfx = plsc.cumsum(cnt)
    total = pfx[15]

    for l in range(16):
        nl = cnt[l]
        st = pfx[l] - nl

        def mv(tt, carry, l=l, nl=nl, st=st):
            pos = 16 * tt + iota
            m = pos < nl
            sv = pend_s[pl.ds(l * CAPL + 16 * tt, 16)]
            lv = pend_l[pl.ds(l * CAPL + 16 * tt, 16)]
            plsc.store_scatter(msrc, [st + pos], sv, mask=m)
            plsc.store_scatter(mloc, [st + pos], lv, mask=m)
            return carry

        lax.fori_loop(0, (nl + 15) // 16, mv, 0)

    for m in range(GBLK // 16):
        plsc.store_scatter(msrc, [total + 16 * m + iota],
                           jnp.zeros((16,), jnp.int32))
        plsc.store_scatter(mloc, [total + 16 * m + iota],
                           jnp.full((16,), RPT, jnp.int32))

    nb = total // GBLK + 1

    def accum(b, buf):
        for q in range(GBLK // 16):
            ldvec = mloc[pl.ds(b * GBLK + 16 * q, 16)]
            for r in range(16):
                ld = ldvec[r]
                for k in range(f // 16):
                    plsc.addupdate(acc.at[pl.ds(ld * f + 16 * k, 16)],
                                   buf[16 * q + r, pl.ds(16 * k, 16)])

    _sc_fire(u_hbm, msrc, 0, rows_a, sem_a)

    def pairs(i, carry):
        b0 = 2 * i

        @pl.when(b0 + 1 < nb)
        def _():
            _sc_fire(u_hbm, msrc, b0 + 1, rows_b, sem_b)
        _sc_wait(u_hbm, rows_a, sem_a)
        accum(b0, rows_a)

        @pl.when(b0 + 2 < nb)
        def _():
            _sc_fire(u_hbm, msrc, b0 + 2, rows_a, sem_a)

        @pl.when(b0 + 1 < nb)
        def _():
            _sc_wait(u_hbm, rows_b, sem_b)
            accum(b0 + 1, rows_b)
        return carry

    lax.fori_loop(0, (nb + 1) // 2, pairs, 0)

    pltpu.sync_copy(acc.at[pl.ds(0, RPT * f)],
                    out_hbm.at[pl.ds(base * f, RPT * f)])


def _sc_agg(u, sp, dp, f):
    nblk = sp.shape[0] // EBLK
    mesh = plsc.VectorSubcoreMesh(core_axis_name="c", subcore_axis_name="s")
    k = pl.kernel(
        functools.partial(_sc_agg_body, nblk, f),
        out_type=jax.ShapeDtypeStruct((NDP * f,), F32),
        mesh=mesh,
        compiler_params=pltpu.CompilerParams(needs_layout_passes=False),
        scratch_types=[
            pltpu.VMEM((EBLK,), jnp.int32),
            pltpu.VMEM((EBLK,), jnp.int32),
            pltpu.VMEM((16 * CAPL,), jnp.int32),
            pltpu.VMEM((16 * CAPL,), jnp.int32),
            pltpu.VMEM((MCAP,), jnp.int32),
            pltpu.VMEM((MCAP,), jnp.int32),
            pltpu.VMEM((GBLK, f), F32),
            pltpu.VMEM((GBLK, f), F32),
            pltpu.VMEM((ACCR * f,), F32),
            pltpu.SemaphoreType.DMA,
            pltpu.SemaphoreType.DMA,
        ],
    )
    return k(u, sp, dp).reshape(NDP, f)


def _edge_layout(s, d):
    e = s.shape[0]
    tot = -(-e // EBLK) * EBLK
    sp = jnp.zeros((tot,), jnp.int32).at[:e].set(s)
    dp = jnp.full((tot,), NDP, jnp.int32).at[:e].set(d)
    return sp, dp


SEGR = 224
SEGF = 384


def _sc_segmax_body(x_hbm, b_hbm, out_hbm, bb, rbuf, acc, sem):
    w = lax.axis_index("c") * 16 + lax.axis_index("s")
    ninf = jnp.full((16,), -jnp.inf, F32)

    def zacc(m, carry):
        for k in range(16):
            acc[pl.ds(m * 256 + 16 * k, 16)] = ninf
        return carry

    lax.fori_loop(0, (SEGR * SEGF + 256) // 256, zacc, 0)

    pltpu.sync_copy(b_hbm.at[pl.ds(w * RPT, RPT)], bb)

    def chunk(m, carry):
        pltpu.sync_copy(x_hbm.at[pl.ds(w * RPT + 16 * m, 16)], rbuf)
        bv = bb[pl.ds(16 * m, 16)]
        for r in range(16):
            sb = bv[r] * SEGF
            for k in range(SEGF // 16):
                cur = acc[pl.ds(sb + 16 * k, 16)]
                acc[pl.ds(sb + 16 * k, 16)] = jnp.maximum(
                    cur, rbuf[r, pl.ds(16 * k, 16)])
        return carry

    lax.fori_loop(0, RPT // 16, chunk, 0)

    pltpu.sync_copy(acc.at[pl.ds(0, SEGR * SEGF)],
                    out_hbm.at[pl.ds(w * SEGR * SEGF, SEGR * SEGF)])


def _sc_segmax(x4, batch_pad):
    mesh = plsc.VectorSubcoreMesh(core_axis_name="c", subcore_axis_name="s")
    k = pl.kernel(
        _sc_segmax_body,
        out_type=jax.ShapeDtypeStruct((32 * SEGR * SEGF,), F32),
        mesh=mesh,
        compiler_params=pltpu.CompilerParams(needs_layout_passes=False),
        scratch_types=[
            pltpu.VMEM((RPT,), jnp.int32),
            pltpu.VMEM((16, SEGF), F32),
            pltpu.VMEM((SEGR * SEGF + 256,), F32),
            pltpu.SemaphoreType.DMA,
        ],
    )
    return k(x4, batch_pad).reshape(32, SEGR, SEGF)


CROWS = 16


def _sc_counts_scan(idx_hbm, nblk, stage, pend, lane_base, to_local):
    def scan(j, cnt):
        pltpu.sync_copy(idx_hbm.at[pl.ds(j * EBLK, EBLK)], stage)
        for k in range(EBLK // 16):
            val, ok = to_local(stage[pl.ds(16 * k, 16)], k)
            idx = lane_base + jnp.minimum(cnt, CAPL - 1)
            plsc.store_scatter(pend, [idx], val, mask=ok)
            cnt = cnt + jnp.where(ok, 1, 0)
        return cnt

    return lax.fori_loop(0, nblk, scan, jnp.zeros((16,), jnp.int32))


def _sc_counts_merge(pend, merged, cnt, dump, iota):
    pfx = plsc.cumsum(cnt)
    total = pfx[15]
    for l in range(16):
        nl = cnt[l]
        st = pfx[l] - nl

        def mv(tt, carry, l=l, nl=nl, st=st):
            pos = 16 * tt + iota
            m = pos < nl
            lv = pend[pl.ds(l * CAPL + 16 * tt, 16)]
            plsc.store_scatter(merged, [st + pos], lv, mask=m)
            return carry

        lax.fori_loop(0, (nl + 15) // 16, mv, 0)
    plsc.store_scatter(merged, [total + iota],
                       jnp.full((16,), dump, jnp.int32))
    return total


def _sc_counts_apply(merged, total, acc, e0):
    def app(t, carry):
        ldvec = merged[pl.ds(16 * t, 16)]
        for r in range(16):
            plsc.addupdate(acc.at[pl.ds(ldvec[r], 16)], e0)
        return carry

    lax.fori_loop(0, total // 16 + 1, app, 0)


def _sc_counts_body(ndblk, neblk, dd_hbm, es_hbm, ed_hbm, deg_hbm, c_hbm,
                    stage, stage2, pend, merged, dacc, cacc, sem):
    w = lax.axis_index("c") * 16 + lax.axis_index("s")
    iota = lax.iota(jnp.int32, 16)
    lane_base = iota * CAPL
    zvec = jnp.zeros((16,), F32)
    e0 = jnp.where(iota == 0, 1.0, 0.0).astype(F32)

    for k in range(RPT // 16 + 1):
        dacc[pl.ds(16 * k, 16)] = zvec

    def zc(m, carry):
        for k in range(16):
            cacc[pl.ds(m * 256 + 16 * k, 16)] = zvec
        return carry

    lax.fori_loop(0, (CROWS * 512 + 256) // 256, zc, 0)

    base = w * RPT

    def loc_deg(dv, k):
        loc = dv - base
        return loc, (loc >= 0) & (loc < RPT)

    cnt = _sc_counts_scan(dd_hbm, ndblk, stage, pend, lane_base, loc_deg)
    total = _sc_counts_merge(pend, merged, cnt, RPT, iota)
    _sc_counts_apply(merged, total, dacc, e0)
    pltpu.sync_copy(dacc.at[pl.ds(0, RPT)], deg_hbm.at[pl.ds(base, RPT)])

    cbase = w * CROWS

    def loc_c(dv, k):
        loc = dv - cbase
        ok = (loc >= 0) & (loc < CROWS)
        sv = stage2[pl.ds(16 * k, 16)]
        return loc * 512 + sv, ok

    def scan2(j, cnt2):
        pltpu.sync_copy(ed_hbm.at[pl.ds(j * EBLK, EBLK)], stage)
        pltpu.sync_copy(es_hbm.at[pl.ds(j * EBLK, EBLK)], stage2)
        for k in range(EBLK // 16):
            val, ok = loc_c(stage[pl.ds(16 * k, 16)], k)
            idx = lane_base + jnp.minimum(cnt2, CAPL - 1)
            plsc.store_scatter(pend, [idx], val, mask=ok)
            cnt2 = cnt2 + jnp.where(ok, 1, 0)
        return cnt2

    cnt2 = lax.fori_loop(0, neblk, scan2, jnp.zeros((16,), jnp.int32))
    total2 = _sc_counts_merge(pend, merged, cnt2, CROWS * 512, iota)
    _sc_counts_apply(merged, total2, cacc, e0)
    pltpu.sync_copy(cacc.at[pl.ds(0, CROWS * 512)],
                    c_hbm.at[pl.ds(cbase * 512, CROWS * 512)])


def _sc_counts(dd, es, ed):
    ndblk = dd.shape[0] // EBLK
    neblk = es.shape[0] // EBLK
    mesh = plsc.VectorSubcoreMesh(core_axis_name="c", subcore_axis_name="s")
    k = pl.kernel(
        functools.partial(_sc_counts_body, ndblk, neblk),
        out_type=[jax.ShapeDtypeStruct((NDP,), F32),
                  jax.ShapeDtypeStruct((512 * 512,), F32)],
        mesh=mesh,
        compiler_params=pltpu.CompilerParams(needs_layout_passes=False),
        scratch_types=[
            pltpu.VMEM((EBLK,), jnp.int32),
            pltpu.VMEM((EBLK,), jnp.int32),
            pltpu.VMEM((16 * CAPL,), jnp.int32),
            pltpu.VMEM((16 * CAPL + 2 * GBLK,), jnp.int32),
            pltpu.VMEM((RPT + 32,), F32),
            pltpu.VMEM((CROWS * 512 + 256 + 32,), F32),
            pltpu.SemaphoreType.DMA,
        ],
    )
    deg, cf = k(dd, es, ed)
    return deg, cf.reshape(512, 512)


def _middle_body(pool_ref, x1_ref, c_ref, wfc_ref, bfc_ref,
                 wg_ref, bg_ref, sel_ref, fc1w_ref, fc1b_ref,
                 w0t_ref, w0b_ref, a_ref, b_ref):
    rr = jax.lax.broadcasted_iota(jnp.int32, (512, 512), 0)
    cc = jax.lax.broadcasted_iota(jnp.int32, (512, 512), 1)
    C = c_ref[...] + jnp.where((rr == cc) & (rr < N_NODES), 1.0, 0.0)
    deg = jnp.sum(C, axis=1)
    dinv = jnp.where(deg > 0, jax.lax.rsqrt(deg), 0.0)
    Adj = dinv[:, None] * C * dinv[None, :]

    pmax = jnp.max(pool_ref[...], axis=0)
    pooled = jnp.where(jnp.isfinite(pmax), pmax, 0.0)
    gfeat = jax.nn.relu(_bdot(pooled, wfc_ref[...])
                        + bfc_ref[...][None, :])
    rows = jax.lax.broadcasted_iota(jnp.int32, (512, 1), 0)
    xcat = x1_ref[...] + jnp.where(rows < N_DRUGS, _pad_rows(gfeat, 512), 0.0)

    sel = sel_ref[...][:, None]
    xsel = jnp.zeros((512, 512), F32)
    for l in range(3):
        xl = jax.nn.relu(_bdot(Adj.astype(F32), _bdot(xcat, wg_ref[l]))
                         + bg_ref[l][None, :])
        xsel = xsel + jnp.where(sel == l, xl, 0.0)
    xf = jax.nn.relu(_bdot(xsel, fc1w_ref[...]) + fc1b_ref[...][None, :])
    a_ref[...] = _bdot(xf, w0t_ref[0]) + _bdot(xcat, w0t_ref[1])
    b_ref[...] = _bdot(xf, w0b_ref[0]) + _bdot(xcat, w0b_ref[1])


def _pad_rows(a, n):
    return jnp.pad(a, ((0, n - a.shape[0]), (0, 0)))


def _middle(pool, x1p, C, wfc, bfc, wg, bg, sel, fc1w, fc1b, w0t, w0b):
    fs = _full_spec
    return pl.pallas_call(
        _middle_body,
        grid=(1,),
        in_specs=[pl.BlockSpec((32, 224, 384), lambda i: (0, 0, 0)),
                  fs(512, 512), fs(512, 512), fs(384, 512),
                  _vec_spec(512), pl.BlockSpec((3, 512, 512), lambda i: (0, 0, 0)),
                  pl.BlockSpec((3, 512), lambda i: (0, 0)),
                  pl.BlockSpec((512,), lambda i: (0,)), fs(512, 512),
                  _vec_spec(512), pl.BlockSpec((2, 512, 512), lambda i: (0, 0, 0)),
                  pl.BlockSpec((2, 512, 512), lambda i: (0, 0, 0))],
        out_specs=[fs(512, 512), fs(512, 512)],
        out_shape=[jax.ShapeDtypeStruct((512, 512), F32),
                   jax.ShapeDtypeStruct((512, 512), F32)],
    )(pool, x1p, C, wfc, bfc, wg, bg, sel, fc1w, fc1b, w0t, w0b)


def _mlp_tail(z0, w1_ref, b1_ref, w2_ref, b2_ref, wl_ref, bl_ref):
    h = jax.nn.relu(z0)
    h = jax.nn.relu(_bdot(h, w1_ref[...]) + b1_ref[...][None, :])
    h = jax.nn.relu(_bdot(h, w2_ref[...]) + b2_ref[...][None, :])
    logit = jnp.sum(h * wl_ref[...][None, :], axis=1) + bl_ref[0]
    return jax.nn.sigmoid(logit)


def _out2_body(a2_ref, b2_ref, b0_ref, w1_ref, b1_ref, w2_ref, b2w_ref,
               wl_ref, bl_ref, o_ref, *, bi):
    z0 = (b2_ref[...][:, None, :] + a2_ref[...][None, :, :]
          + b0_ref[...][None, None, :]).reshape(bi * 272, 512)
    o_ref[...] = _mlp_tail(z0, w1_ref, b1_ref, w2_ref, b2w_ref,
                           wl_ref, bl_ref).reshape(bi, 272)


def _out2(a2, b2, b0, w1, b1, w2, b2w, wl, bl, bi=16):
    nblk = 224 // bi
    return pl.pallas_call(
        functools.partial(_out2_body, bi=bi),
        grid=(nblk,),
        in_specs=[_full_spec(272, 512), pl.BlockSpec((bi, 512), lambda i: (i, 0)),
                  _vec_spec(512), _full_spec(512, 512), _vec_spec(512),
                  _full_spec(512, 512), _vec_spec(512), _vec_spec(512),
                  _vec_spec(8)],
        out_specs=pl.BlockSpec((bi, 272), lambda i: (i, 0)),
        out_shape=jax.ShapeDtypeStruct((224, 272), F32),
    )(a2, b2, b0, w1, b1, w2, b2w, wl, bl)


def _out1_body(rr_ref, dd_ref, a_ref, b_ref, b0_ref,
               w1_ref, b1_ref, w2_ref, b2_ref, wl_ref, bl_ref, o_ref):
    cols = jax.lax.broadcasted_iota(jnp.int32, (1024, 512), 1)
    ohr = (rr_ref[...][:, None] == cols).astype(F32)
    ohd = (dd_ref[...][:, None] == cols).astype(F32)
    z0 = (_bdot(ohr, a_ref[...]) + _bdot(ohd, b_ref[...])
          + b0_ref[...][None, :])
    o_ref[...] = _mlp_tail(z0, w1_ref, b1_ref, w2_ref, b2_ref,
                           wl_ref, bl_ref)


def _out1(rr, ddx, a, b, b0, w1, b1, w2, b2, wl, bl):
    return pl.pallas_call(
        _out1_body,
        grid=(8,),
        in_specs=[pl.BlockSpec((1024,), lambda i: (i,)),
                  pl.BlockSpec((1024,), lambda i: (i,)),
                  _full_spec(512, 512), _full_spec(512, 512), _vec_spec(512),
                  _full_spec(512, 512), _vec_spec(512), _full_spec(512, 512),
                  _vec_spec(512), _vec_spec(512), _vec_spec(8)],
        out_specs=pl.BlockSpec((1024,), lambda i: (i,)),
        out_shape=jax.ShapeDtypeStruct((8192,), F32),
    )(rr, ddx, a, b, b0, w1, b1, w2, b2, wl, bl)


def kernel(x1, edges, hop, edges2, drug_x, drug_edge_index, drug_batch, params):
    p = params
    s, dd = drug_edge_index[0], drug_edge_index[1]

    inv = 1.0 / np.sqrt(1.0 + BN_EPS)
    g0, g1, g2 = p['bn_g0'] * inv, p['bn_g1'] * inv, p['bn_g2'] * inv
    w1p = _pad2(g0[:, None] * p['d_W1'], 512, 512)
    b1p = _pad1(p['bn_b0'] @ p['d_W1'] + p['d_b1'], 512)
    w2p = _pad2(g1[:, None] * p['d_W2'], 512, 512)
    b2p = _pad1(p['bn_b1'] @ p['d_W2'] + p['d_b2'], 512)
    wlp = _pad1((g2[:, None] * p['d_Wl'])[:, 0], 512)
    blp = _pad1(p['bn_b2'] @ p['d_Wl'] + p['d_bl'], 8)
    b0p = _pad1(p['d_b0'], 512)

    gw1 = _pad2(p['g_W1'], 128, 128)
    gw2 = _pad2(p['g_W2'], 128, 256)
    gw3 = _pad2(p['g_W3'], 256, 384)
    gwfc = _pad2(p['g_Wfc'], 384, 512)
    gb1 = _pad1(p['g_b1'], 128)
    gb2 = _pad1(p['g_b2'], 256)
    gb3 = _pad1(p['g_b3'], 384)
    gbfc = _pad1(p['g_bfc'], 512)
    wg = jnp.stack([_pad2(p['W_g%d' % l], 512, 512) for l in range(3)])
    bg = jnp.stack([_pad1(p['b_g%d' % l], 512) for l in range(3)])
    fc1w = _pad2(p['fc1_W'], 512, 512)
    fc1b = _pad1(p['fc1_b'], 512)
    w0t = jnp.stack([_pad2(p['d_W0'][:489], 512, 512),
                     _pad2(p['d_W0'][489:978], 512, 512)])
    w0b = jnp.stack([_pad2(p['d_W0'][978:978 + 489], 512, 512),
                     _pad2(p['d_W0'][978 + 489:], 512, 512)])

    spi, dpi = _edge_layout(s, dd)
    esp, edp = _edge_layout(edges[0], edges[1])
    deg, C = _sc_counts(dpi, esp, edp)
    dinv = jnp.where(jnp.arange(NDP) < ND, (deg + 1.0) ** -0.5, 0.0)

    xq = _pad2(drug_x, NDP, 128)
    v1 = _drug_scale(xq, dinv, 128)
    y1 = _sc_agg(v1, spi, dpi, 128)
    v2, _ = _drug_mid(y1, v1, dinv, gb1, gw1, 128, 128)
    y2 = _sc_agg(v2, spi, dpi, 128)
    v3, _ = _drug_mid(y2, v2, dinv, gb2, gw2, 128, 256)
    y3 = _sc_agg(v3, spi, dpi, 256)
    x4 = _drug_fin(y3, v3, dinv, gb3, gw3, 256, 384)

    batch_pad = jnp.full((NDP,), N_DRUGS, jnp.int32).at[:ND].set(drug_batch)
    pool = _sc_segmax(x4, batch_pad)

    x1p = _pad2(x1, 512, 512)
    sel = _pad1(jnp.where(hop == 0, 2, hop - 1).astype(jnp.int32), 512)
    A, B = _middle(pool, x1p, C, gwfc, gbfc, wg, bg, sel, fc1w, fc1b,
                   w0t, w0b)

    a2 = _pad_rows(A[N_DRUGS:N_NODES], 272)
    b2 = B[:224]
    out2 = _out2(a2, b2, b0p, w1p, b1p, w2p, b2p, wlp, blp)[:N_DRUGS, :271]

    out1 = _out1(edges2[1], edges2[0], A, B, b0p,
                 w1p, b1p, w2p, b2p, wlp, blp)

    return out1, out2


# speedup vs baseline: 2.2876x; 1.1508x over previous
import functools

import jax
import jax.numpy as jnp
import numpy as np
from jax import lax
from jax.experimental import pallas as pl
from jax.experimental.pallas import tpu as pltpu
from jax.experimental.pallas import tpu_sc as plsc

N_DRUGS = 218
N_NODES = 489
BN_EPS = 1e-5
F32 = jnp.float32
BF16 = jnp.bfloat16

ND = 6540
NDP = 6656
ROWB = 512


def _pad2(a, r, c):
    return jnp.zeros((r, c), a.dtype).at[: a.shape[0], : a.shape[1]].set(a)


def _pad1(a, n):
    return jnp.zeros((n,), a.dtype).at[: a.shape[0]].set(a)


def _bdot(a, b):
    return jax.lax.dot(a.astype(BF16), b.astype(BF16),
                       preferred_element_type=F32)


def _drug_scale_body(x_ref, dinv_ref, v_ref):
    v_ref[...] = dinv_ref[...][:, None] * x_ref[...]


def _drug_mid_body(y_ref, v_ref, dinv_ref, b_ref, w_ref, vo_ref, x_ref):
    dinv = dinv_ref[...][:, None]
    x = jax.nn.relu(dinv * _bdot(y_ref[...] + v_ref[...], w_ref[...])
                    + b_ref[...][None, :])
    x_ref[...] = x
    vo_ref[...] = dinv * x


def _drug_fin_body(y_ref, v_ref, dinv_ref, b_ref, w_ref, x_ref):
    dinv = dinv_ref[...][:, None]
    x_ref[...] = jax.nn.relu(dinv * _bdot(y_ref[...] + v_ref[...], w_ref[...])
                             + b_ref[...][None, :])


def _row_spec(c):
    return pl.BlockSpec((ROWB, c), lambda i: (i, 0))


def _vec_spec(n):
    return pl.BlockSpec((n,), lambda i: (0,))


def _full_spec(r, c):
    return pl.BlockSpec((r, c), lambda i: (0, 0))


def _drug_scale(x, dinv, fin):
    return pl.pallas_call(
        _drug_scale_body,
        grid=(NDP // ROWB,),
        in_specs=[_row_spec(fin), pl.BlockSpec((ROWB,), lambda i: (i,))],
        out_specs=_row_spec(fin),
        out_shape=jax.ShapeDtypeStruct((NDP, fin), F32),
    )(x, dinv)


def _drug_mid(y, v, dinv, b, w, fin, fout):
    return pl.pallas_call(
        _drug_mid_body,
        grid=(NDP // ROWB,),
        in_specs=[_row_spec(fin), _row_spec(fin), pl.BlockSpec((ROWB,), lambda i: (i,)),
                  _vec_spec(fout), _full_spec(fin, fout)],
        out_specs=[_row_spec(fout), _row_spec(fout)],
        out_shape=[jax.ShapeDtypeStruct((NDP, fout), F32),
                   jax.ShapeDtypeStruct((NDP, fout), F32)],
    )(y, v, dinv, b, w)


def _drug_fin(y, v, dinv, b, w, fin, fout):
    return pl.pallas_call(
        _drug_fin_body,
        grid=(NDP // ROWB,),
        in_specs=[_row_spec(fin), _row_spec(fin), pl.BlockSpec((ROWB,), lambda i: (i,)),
                  _vec_spec(fout), _full_spec(fin, fout)],
        out_specs=_row_spec(fout),
        out_shape=jax.ShapeDtypeStruct((NDP, fout), F32),
    )(y, v, dinv, b, w)


RPT = 208
ACCR = RPT + 8
EBLK = 1024
GBLK = 32
CAPL = 128
MCAP = 16 * CAPL + 2 * GBLK


def _sc_fire(u_hbm, msrc, b, buf, sem):
    pltpu.async_copy(u_hbm.at[msrc.at[pl.ds(b * GBLK, GBLK)]], buf, sem)


def _sc_wait(u_hbm, buf, sem):
    pltpu.make_async_copy(u_hbm.at[pl.ds(0, GBLK)], buf, sem).wait()


def _sc_agg_body(ne, f, u_hbm, sp_hbm, dp_hbm, out_hbm,
                 sblk, dblk, pend_s, pend_l, msrc, mloc,
                 rows_a, rows_b, acc, sem_a, sem_b):
    w = lax.axis_index("c") * 16 + lax.axis_index("s")
    base = w * RPT
    iota = lax.iota(jnp.int32, 16)
    lane_base = iota * CAPL

    pltpu.async_copy(sp_hbm, sblk, sem_a)
    pltpu.async_copy(dp_hbm, dblk, sem_b)

    zvec = jnp.zeros((16,), F32)

    def zacc(m, carry):
        for k in range(16):
            acc[pl.ds(m * 256 + 16 * k, 16)] = zvec
        return carry

    lax.fori_loop(0, ACCR * f // 256, zacc, 0)
    pltpu.make_async_copy(sp_hbm, sblk, sem_a).wait()
    pltpu.make_async_copy(dp_hbm, dblk, sem_b).wait()

    def scan(j, cnt):
        dv = dblk[pl.ds(16 * j, 16)]
        loc = dv - base
        ok = (loc >= 0) & (loc < RPT)
        idx = lane_base + jnp.minimum(cnt, CAPL - 1)
        plsc.store_scatter(pend_s, [idx], sblk[pl.ds(16 * j, 16)], mask=ok)
        plsc.store_scatter(pend_l, [idx], loc, mask=ok)
        return cnt + jnp.where(ok, 1, 0)

    cnt = lax.fori_loop(0, ne // 16, scan, jnp.zeros((16,), jnp.int32))

    pfx = plsc.cumsum(cnt)
    total = pfx[15]

    for l in range(16):
        nl = cnt[l]
        st = pfx[l] - nl

        def mv(tt, carry, l=l, nl=nl, st=st):
            pos = 16 * tt + iota
            m = pos < nl
            sv = pend_s[pl.ds(l * CAPL + 16 * tt, 16)]
            lv = pend_l[pl.ds(l * CAPL + 16 * tt, 16)]
            plsc.store_scatter(msrc, [st + pos], sv, mask=m)
            plsc.store_scatter(mloc, [st + pos], lv, mask=m)
            return carry

        lax.fori_loop(0, (nl + 15) // 16, mv, 0)

    for m in range(GBLK // 16):
        plsc.store_scatter(msrc, [total + 16 * m + iota],
                           jnp.zeros((16,), jnp.int32))
        plsc.store_scatter(mloc, [total + 16 * m + iota],
                           jnp.full((16,), RPT, jnp.int32))

    nb = total // GBLK + 1

    def accum(b, buf):
        for q in range(GBLK // 16):
            ldvec = mloc[pl.ds(b * GBLK + 16 * q, 16)]
            for r in range(16):
                ld = ldvec[r]
                for k in range(f // 16):
                    plsc.addupdate(acc.at[pl.ds(ld * f + 16 * k, 16)],
                                   buf[16 * q + r, pl.ds(16 * k, 16)])

    _sc_fire(u_hbm, msrc, 0, rows_a, sem_a)

    def pairs(i, carry):
        b0 = 2 * i

        @pl.when(b0 + 1 < nb)
        def _():
            _sc_fire(u_hbm, msrc, b0 + 1, rows_b, sem_b)
        _sc_wait(u_hbm, rows_a, sem_a)
        accum(b0, rows_a)

        @pl.when(b0 + 2 < nb)
        def _():
            _sc_fire(u_hbm, msrc, b0 + 2, rows_a, sem_a)

        @pl.when(b0 + 1 < nb)
        def _():
            _sc_wait(u_hbm, rows_b, sem_b)
            accum(b0 + 1, rows_b)
        return carry

    lax.fori_loop(0, (nb + 1) // 2, pairs, 0)

    pltpu.sync_copy(acc.at[pl.ds(0, RPT * f)],
                    out_hbm.at[pl.ds(base * f, RPT * f)])


def _sc_agg(u, sp, dp, f):
    ne = sp.shape[0]
    mesh = plsc.VectorSubcoreMesh(core_axis_name="c", subcore_axis_name="s")
    k = pl.kernel(
        functools.partial(_sc_agg_body, ne, f),
        out_type=jax.ShapeDtypeStruct((NDP * f,), F32),
        mesh=mesh,
        compiler_params=pltpu.CompilerParams(needs_layout_passes=False),
        scratch_types=[
            pltpu.VMEM((ne,), jnp.int32),
            pltpu.VMEM((ne,), jnp.int32),
            pltpu.VMEM((16 * CAPL,), jnp.int32),
            pltpu.VMEM((16 * CAPL,), jnp.int32),
            pltpu.VMEM((MCAP,), jnp.int32),
            pltpu.VMEM((MCAP,), jnp.int32),
            pltpu.VMEM((GBLK, f), F32),
            pltpu.VMEM((GBLK, f), F32),
            pltpu.VMEM((ACCR * f,), F32),
            pltpu.SemaphoreType.DMA,
            pltpu.SemaphoreType.DMA,
        ],
    )
    return k(u, sp, dp).reshape(NDP, f)


def _edge_layout(s, d):
    e = s.shape[0]
    tot = -(-e // EBLK) * EBLK
    sp = jnp.zeros((tot,), jnp.int32).at[:e].set(s)
    dp = jnp.full((tot,), NDP, jnp.int32).at[:e].set(d)
    return sp, dp


SEGR = 224
SEGF = 384


def _sx_fire(x_hbm, w, m, buf, sem):
    pltpu.async_copy(x_hbm.at[pl.ds(w * RPT + 16 * m, 16)], buf, sem)


def _sx_wait(x_hbm, buf, sem):
    pltpu.make_async_copy(x_hbm.at[pl.ds(0, 16)], buf, sem).wait()


def _sc_segmax_body(x_hbm, b_hbm, out_hbm, bb, rb_a, rb_b, acc,
                    sem_a, sem_b, sem_c):
    w = lax.axis_index("c") * 16 + lax.axis_index("s")
    ninf = jnp.full((16,), -jnp.inf, F32)
    nchunk = RPT // 16

    _sx_fire(x_hbm, w, 0, rb_a, sem_a)
    pltpu.async_copy(b_hbm.at[pl.ds(w * RPT, RPT)], bb, sem_c)

    def zacc(m, carry):
        for k in range(16):
            acc[pl.ds(m * 256 + 16 * k, 16)] = ninf
        return carry

    lax.fori_loop(0, (SEGR * SEGF + 256) // 256, zacc, 0)
    pltpu.make_async_copy(b_hbm.at[pl.ds(0, RPT)], bb, sem_c).wait()

    def chunk(m, buf):
        bv = bb[pl.ds(16 * m, 16)]
        for r in range(16):
            sb = bv[r] * SEGF
            for k in range(SEGF // 16):
                cur = acc[pl.ds(sb + 16 * k, 16)]
                acc[pl.ds(sb + 16 * k, 16)] = jnp.maximum(
                    cur, buf[r, pl.ds(16 * k, 16)])

    def pairs(i, carry):
        m0 = 2 * i

        @pl.when(m0 + 1 < nchunk)
        def _():
            _sx_fire(x_hbm, w, m0 + 1, rb_b, sem_b)
        _sx_wait(x_hbm, rb_a, sem_a)
        chunk(m0, rb_a)

        @pl.when(m0 + 2 < nchunk)
        def _():
            _sx_fire(x_hbm, w, m0 + 2, rb_a, sem_a)

        @pl.when(m0 + 1 < nchunk)
        def _():
            _sx_wait(x_hbm, rb_b, sem_b)
            chunk(m0 + 1, rb_b)
        return carry

    lax.fori_loop(0, (nchunk + 1) // 2, pairs, 0)

    pltpu.sync_copy(acc.at[pl.ds(0, SEGR * SEGF)],
                    out_hbm.at[pl.ds(w * SEGR * SEGF, SEGR * SEGF)])


def _sc_segmax(x4, batch_pad):
    mesh = plsc.VectorSubcoreMesh(core_axis_name="c", subcore_axis_name="s")
    k = pl.kernel(
        _sc_segmax_body,
        out_type=jax.ShapeDtypeStruct((32 * SEGR * SEGF,), F32),
        mesh=mesh,
        compiler_params=pltpu.CompilerParams(needs_layout_passes=False),
        scratch_types=[
            pltpu.VMEM((RPT,), jnp.int32),
            pltpu.VMEM((16, SEGF), F32),
            pltpu.VMEM((16, SEGF), F32),
            pltpu.VMEM((SEGR * SEGF + 256,), F32),
            pltpu.SemaphoreType.DMA,
            pltpu.SemaphoreType.DMA,
            pltpu.SemaphoreType.DMA,
        ],
    )
    return k(x4, batch_pad).reshape(32, SEGR, SEGF)


CROWS = 16


def _sc_counts_merge(pend, merged, cnt, dump, iota):
    pfx = plsc.cumsum(cnt)
    total = pfx[15]
    for l in range(16):
        nl = cnt[l]
        st = pfx[l] - nl

        def mv(tt, carry, l=l, nl=nl, st=st):
            pos = 16 * tt + iota
            m = pos < nl
            lv = pend[pl.ds(l * CAPL + 16 * tt, 16)]
            plsc.store_scatter(merged, [st + pos], lv, mask=m)
            return carry

        lax.fori_loop(0, (nl + 15) // 16, mv, 0)
    plsc.store_scatter(merged, [total + iota],
                       jnp.full((16,), dump, jnp.int32))
    return total


def _sc_counts_apply(merged, total, acc, e0):
    def app(t, carry):
        ldvec = merged[pl.ds(16 * t, 16)]
        for r in range(16):
            plsc.addupdate(acc.at[pl.ds(ldvec[r], 16)], e0)
        return carry

    lax.fori_loop(0, total // 16 + 1, app, 0)


def _sc_counts_body(nd, ne, dd_hbm, es_hbm, ed_hbm, deg_hbm, c_hbm,
                    stage, stage2, pend, merged, dacc, cacc, sem):
    w = lax.axis_index("c") * 16 + lax.axis_index("s")
    iota = lax.iota(jnp.int32, 16)
    lane_base = iota * CAPL
    zvec = jnp.zeros((16,), F32)
    e0 = jnp.where(iota == 0, 1.0, 0.0).astype(F32)

    pltpu.async_copy(dd_hbm, stage.at[pl.ds(0, nd)], sem)
    pltpu.async_copy(es_hbm, stage2.at[pl.ds(0, ne)], sem)

    for k in range(RPT // 16 + 1):
        dacc[pl.ds(16 * k, 16)] = zvec

    def zc(m, carry):
        for k in range(16):
            cacc[pl.ds(m * 256 + 16 * k, 16)] = zvec
        return carry

    lax.fori_loop(0, (CROWS * 512 + 256) // 256, zc, 0)
    pltpu.make_async_copy(dd_hbm, stage.at[pl.ds(0, nd)], sem).wait()
    pltpu.make_async_copy(es_hbm, stage2.at[pl.ds(0, ne)], sem).wait()

    base = w * RPT

    def scan1(j, cnt):
        loc = stage[pl.ds(16 * j, 16)] - base
        ok = (loc >= 0) & (loc < RPT)
        idx = lane_base + jnp.minimum(cnt, CAPL - 1)
        plsc.store_scatter(pend, [idx], loc, mask=ok)
        return cnt + jnp.where(ok, 1, 0)

    cnt = lax.fori_loop(0, nd // 16, scan1, jnp.zeros((16,), jnp.int32))
    total = _sc_counts_merge(pend, merged, cnt, RPT, iota)
    _sc_counts_apply(merged, total, dacc, e0)
    pltpu.sync_copy(dacc.at[pl.ds(0, RPT)], deg_hbm.at[pl.ds(base, RPT)])

    pltpu.sync_copy(ed_hbm, stage.at[pl.ds(0, ne)])
    cbase = w * CROWS

    def scan2(j, cnt2):
        loc = stage[pl.ds(16 * j, 16)] - cbase
        ok = (loc >= 0) & (loc < CROWS)
        sv = stage2[pl.ds(16 * j, 16)]
        idx = lane_base + jnp.minimum(cnt2, CAPL - 1)
        plsc.store_scatter(pend, [idx], loc * 512 + sv, mask=ok)
        return cnt2 + jnp.where(ok, 1, 0)

    cnt2 = lax.fori_loop(0, ne // 16, scan2, jnp.zeros((16,), jnp.int32))
    total2 = _sc_counts_merge(pend, merged, cnt2, CROWS * 512, iota)
    _sc_counts_apply(merged, total2, cacc, e0)
    pltpu.sync_copy(cacc.at[pl.ds(0, CROWS * 512)],
                    c_hbm.at[pl.ds(cbase * 512, CROWS * 512)])


def _sc_counts(dd, es, ed):
    nd = dd.shape[0]
    ne = es.shape[0]
    mesh = plsc.VectorSubcoreMesh(core_axis_name="c", subcore_axis_name="s")
    k = pl.kernel(
        functools.partial(_sc_counts_body, nd, ne),
        out_type=[jax.ShapeDtypeStruct((NDP,), F32),
                  jax.ShapeDtypeStruct((512 * 512,), F32)],
        mesh=mesh,
        compiler_params=pltpu.CompilerParams(needs_layout_passes=False),
        scratch_types=[
            pltpu.VMEM((ne,), jnp.int32),
            pltpu.VMEM((ne,), jnp.int32),
            pltpu.VMEM((16 * CAPL,), jnp.int32),
            pltpu.VMEM((16 * CAPL + 2 * GBLK,), jnp.int32),
            pltpu.VMEM((RPT + 32,), F32),
            pltpu.VMEM((CROWS * 512 + 256 + 32,), F32),
            pltpu.SemaphoreType.DMA,
        ],
    )
    deg, cf = k(dd, es, ed)
    return deg, cf.reshape(512, 512)


def _middle_body(pool_ref, x1_ref, c_ref, wfc_ref, bfc_ref,
                 wg_ref, bg_ref, sel_ref, fc1w_ref, fc1b_ref,
                 w0t_ref, w0b_ref, a_ref, b_ref):
    rr = jax.lax.broadcasted_iota(jnp.int32, (512, 512), 0)
    cc = jax.lax.broadcasted_iota(jnp.int32, (512, 512), 1)
    C = c_ref[...] + jnp.where((rr == cc) & (rr < N_NODES), 1.0, 0.0)
    deg = jnp.sum(C, axis=1)
    dinv = jnp.where(deg > 0, jax.lax.rsqrt(deg), 0.0)
    Adj = dinv[:, None] * C * dinv[None, :]

    pmax = jnp.max(pool_ref[...], axis=0)
    pooled = jnp.where(jnp.isfinite(pmax), pmax, 0.0)
    gfeat = jax.nn.relu(_bdot(pooled, wfc_ref[...])
                        + bfc_ref[...][None, :])
    rows = jax.lax.broadcasted_iota(jnp.int32, (512, 1), 0)
    xcat = x1_ref[...] + jnp.where(rows < N_DRUGS, _pad_rows(gfeat, 512), 0.0)

    sel = sel_ref[...][:, None]
    xsel = jnp.zeros((512, 512), F32)
    for l in range(3):
        xl = jax.nn.relu(_bdot(Adj.astype(F32), _bdot(xcat, wg_ref[l]))
                         + bg_ref[l][None, :])
        xsel = xsel + jnp.where(sel == l, xl, 0.0)
    xf = jax.nn.relu(_bdot(xsel, fc1w_ref[...]) + fc1b_ref[...][None, :])
    a_ref[...] = _bdot(xf, w0t_ref[0]) + _bdot(xcat, w0t_ref[1])
    b_ref[...] = _bdot(xf, w0b_ref[0]) + _bdot(xcat, w0b_ref[1])


def _pad_rows(a, n):
    return jnp.pad(a, ((0, n - a.shape[0]), (0, 0)))


def _middle(pool, x1p, C, wfc, bfc, wg, bg, sel, fc1w, fc1b, w0t, w0b):
    fs = _full_spec
    return pl.pallas_call(
        _middle_body,
        grid=(1,),
        in_specs=[pl.BlockSpec((32, 224, 384), lambda i: (0, 0, 0)),
                  fs(512, 512), fs(512, 512), fs(384, 512),
                  _vec_spec(512), pl.BlockSpec((3, 512, 512), lambda i: (0, 0, 0)),
                  pl.BlockSpec((3, 512), lambda i: (0, 0)),
                  pl.BlockSpec((512,), lambda i: (0,)), fs(512, 512),
                  _vec_spec(512), pl.BlockSpec((2, 512, 512), lambda i: (0, 0, 0)),
                  pl.BlockSpec((2, 512, 512), lambda i: (0, 0, 0))],
        out_specs=[fs(512, 512), fs(512, 512)],
        out_shape=[jax.ShapeDtypeStruct((512, 512), F32),
                   jax.ShapeDtypeStruct((512, 512), F32)],
    )(pool, x1p, C, wfc, bfc, wg, bg, sel, fc1w, fc1b, w0t, w0b)


def _mlp_tail(z0, w1_ref, b1_ref, w2_ref, b2_ref, wl_ref, bl_ref):
    h = jax.nn.relu(z0)
    h = jax.nn.relu(_bdot(h, w1_ref[...]) + b1_ref[...][None, :])
    h = jax.nn.relu(_bdot(h, w2_ref[...]) + b2_ref[...][None, :])
    logit = jnp.sum(h * wl_ref[...][None, :], axis=1) + bl_ref[0]
    return jax.nn.sigmoid(logit)


def _out2_body(a2_ref, b2_ref, b0_ref, w1_ref, b1_ref, w2_ref, b2w_ref,
               wl_ref, bl_ref, o_ref, *, bi):
    z0 = (b2_ref[...][:, None, :] + a2_ref[...][None, :, :]
          + b0_ref[...][None, None, :]).reshape(bi * 272, 512)
    o_ref[...] = _mlp_tail(z0, w1_ref, b1_ref, w2_ref, b2w_ref,
                           wl_ref, bl_ref).reshape(bi, 272)


def _out2(a2, b2, b0, w1, b1, w2, b2w, wl, bl, bi=16):
    nblk = 224 // bi
    return pl.pallas_call(
        functools.partial(_out2_body, bi=bi),
        grid=(nblk,),
        in_specs=[_full_spec(272, 512), pl.BlockSpec((bi, 512), lambda i: (i, 0)),
                  _vec_spec(512), _full_spec(512, 512), _vec_spec(512),
                  _full_spec(512, 512), _vec_spec(512), _vec_spec(512),
                  _vec_spec(8)],
        out_specs=pl.BlockSpec((bi, 272), lambda i: (i, 0)),
        out_shape=jax.ShapeDtypeStruct((224, 272), F32),
    )(a2, b2, b0, w1, b1, w2, b2w, wl, bl)


def _out1_body(rr_ref, dd_ref, a_ref, b_ref, b0_ref,
               w1_ref, b1_ref, w2_ref, b2_ref, wl_ref, bl_ref, o_ref):
    cols = jax.lax.broadcasted_iota(jnp.int32, (1024, 512), 1)
    ohr = (rr_ref[...][:, None] == cols).astype(F32)
    ohd = (dd_ref[...][:, None] == cols).astype(F32)
    z0 = (_bdot(ohr, a_ref[...]) + _bdot(ohd, b_ref[...])
          + b0_ref[...][None, :])
    o_ref[...] = _mlp_tail(z0, w1_ref, b1_ref, w2_ref, b2_ref,
                           wl_ref, bl_ref)


def _out1(rr, ddx, a, b, b0, w1, b1, w2, b2, wl, bl):
    return pl.pallas_call(
        _out1_body,
        grid=(8,),
        in_specs=[pl.BlockSpec((1024,), lambda i: (i,)),
                  pl.BlockSpec((1024,), lambda i: (i,)),
                  _full_spec(512, 512), _full_spec(512, 512), _vec_spec(512),
                  _full_spec(512, 512), _vec_spec(512), _full_spec(512, 512),
                  _vec_spec(512), _vec_spec(512), _vec_spec(8)],
        out_specs=pl.BlockSpec((1024,), lambda i: (i,)),
        out_shape=jax.ShapeDtypeStruct((8192,), F32),
    )(rr, ddx, a, b, b0, w1, b1, w2, b2, wl, bl)


def kernel(x1, edges, hop, edges2, drug_x, drug_edge_index, drug_batch, params):
    p = params
    s, dd = drug_edge_index[0], drug_edge_index[1]

    inv = 1.0 / np.sqrt(1.0 + BN_EPS)
    g0, g1, g2 = p['bn_g0'] * inv, p['bn_g1'] * inv, p['bn_g2'] * inv
    w1p = _pad2(g0[:, None] * p['d_W1'], 512, 512)
    b1p = _pad1(p['bn_b0'] @ p['d_W1'] + p['d_b1'], 512)
    w2p = _pad2(g1[:, None] * p['d_W2'], 512, 512)
    b2p = _pad1(p['bn_b1'] @ p['d_W2'] + p['d_b2'], 512)
    wlp = _pad1((g2[:, None] * p['d_Wl'])[:, 0], 512)
    blp = _pad1(p['bn_b2'] @ p['d_Wl'] + p['d_bl'], 8)
    b0p = _pad1(p['d_b0'], 512)

    gw1 = _pad2(p['g_W1'], 128, 128)
    gw2 = _pad2(p['g_W2'], 128, 256)
    gw3 = _pad2(p['g_W3'], 256, 384)
    gwfc = _pad2(p['g_Wfc'], 384, 512)
    gb1 = _pad1(p['g_b1'], 128)
    gb2 = _pad1(p['g_b2'], 256)
    gb3 = _pad1(p['g_b3'], 384)
    gbfc = _pad1(p['g_bfc'], 512)
    wg = jnp.stack([_pad2(p['W_g%d' % l], 512, 512) for l in range(3)])
    bg = jnp.stack([_pad1(p['b_g%d' % l], 512) for l in range(3)])
    fc1w = _pad2(p['fc1_W'], 512, 512)
    fc1b = _pad1(p['fc1_b'], 512)
    w0t = jnp.stack([_pad2(p['d_W0'][:489], 512, 512),
                     _pad2(p['d_W0'][489:978], 512, 512)])
    w0b = jnp.stack([_pad2(p['d_W0'][978:978 + 489], 512, 512),
                     _pad2(p['d_W0'][978 + 489:], 512, 512)])

    spi, dpi = _edge_layout(s, dd)
    esp, edp = _edge_layout(edges[0], edges[1])
    deg, C = _sc_counts(dpi, esp, edp)
    dinv = jnp.where(jnp.arange(NDP) < ND, (deg + 1.0) ** -0.5, 0.0)

    xq = _pad2(drug_x, NDP, 128)
    v1 = _drug_scale(xq, dinv, 128)
    y1 = _sc_agg(v1, spi, dpi, 128)
    v2, _ = _drug_mid(y1, v1, dinv, gb1, gw1, 128, 128)
    y2 = _sc_agg(v2, spi, dpi, 128)
    v3, _ = _drug_mid(y2, v2, dinv, gb2, gw2, 128, 256)
    y3 = _sc_agg(v3, spi, dpi, 256)
    x4 = _drug_fin(y3, v3, dinv, gb3, gw3, 256, 384)

    batch_pad = jnp.full((NDP,), N_DRUGS, jnp.int32).at[:ND].set(drug_batch)
    pool = _sc_segmax(x4, batch_pad)

    x1p = _pad2(x1, 512, 512)
    sel = _pad1(jnp.where(hop == 0, 2, hop - 1).astype(jnp.int32), 512)
    A, B = _middle(pool, x1p, C, gwfc, gbfc, wg, bg, sel, fc1w, fc1b,
                   w0t, w0b)

    a2 = _pad_rows(A[N_DRUGS:N_NODES], 272)
    b2 = B[:224]
    out2 = _out2(a2, b2, b0p, w1p, b1p, w2p, b2p, wlp, blp)[:N_DRUGS, :271]

    out1 = _out1(edges2[1], edges2[0], A, B, b0p,
                 w1p, b1p, w2p, b2p, wlp, blp)

    return out1, out2


# speedup vs baseline: 2.3015x; 1.0061x over previous
import functools

import jax
import jax.numpy as jnp
import numpy as np
from jax import lax
from jax.experimental import pallas as pl
from jax.experimental.pallas import tpu as pltpu
from jax.experimental.pallas import tpu_sc as plsc

N_DRUGS = 218
N_NODES = 489
BN_EPS = 1e-5
F32 = jnp.float32
BF16 = jnp.bfloat16

ND = 6540
NDP = 6656
ROWB = 512


def _pad2(a, r, c):
    return jnp.zeros((r, c), a.dtype).at[: a.shape[0], : a.shape[1]].set(a)


def _pad1(a, n):
    return jnp.zeros((n,), a.dtype).at[: a.shape[0]].set(a)


def _bdot(a, b):
    return jax.lax.dot(a.astype(BF16), b.astype(BF16),
                       preferred_element_type=F32)


def _drug_scale_body(x_ref, dinv_ref, v_ref):
    v_ref[...] = dinv_ref[...][:, None] * x_ref[...]


def _drug_mid_body(y_ref, v_ref, dinv_ref, b_ref, w_ref, vo_ref, x_ref):
    dinv = dinv_ref[...][:, None]
    x = jax.nn.relu(dinv * _bdot(y_ref[...] + v_ref[...], w_ref[...])
                    + b_ref[...][None, :])
    x_ref[...] = x
    vo_ref[...] = dinv * x


def _drug_fin_body(y_ref, v_ref, dinv_ref, b_ref, w_ref, x_ref):
    dinv = dinv_ref[...][:, None]
    x_ref[...] = jax.nn.relu(dinv * _bdot(y_ref[...] + v_ref[...], w_ref[...])
                             + b_ref[...][None, :])


def _row_spec(c):
    return pl.BlockSpec((ROWB, c), lambda i: (i, 0))


def _vec_spec(n):
    return pl.BlockSpec((n,), lambda i: (0,))


def _full_spec(r, c):
    return pl.BlockSpec((r, c), lambda i: (0, 0))


def _drug_scale(x, dinv, fin):
    return pl.pallas_call(
        _drug_scale_body,
        grid=(NDP // ROWB,),
        in_specs=[_row_spec(fin), pl.BlockSpec((ROWB,), lambda i: (i,))],
        out_specs=_row_spec(fin),
        out_shape=jax.ShapeDtypeStruct((NDP, fin), F32),
    )(x, dinv)


def _drug_mid(y, v, dinv, b, w, fin, fout):
    return pl.pallas_call(
        _drug_mid_body,
        grid=(NDP // ROWB,),
        in_specs=[_row_spec(fin), _row_spec(fin), pl.BlockSpec((ROWB,), lambda i: (i,)),
                  _vec_spec(fout), _full_spec(fin, fout)],
        out_specs=[_row_spec(fout), _row_spec(fout)],
        out_shape=[jax.ShapeDtypeStruct((NDP, fout), F32),
                   jax.ShapeDtypeStruct((NDP, fout), F32)],
    )(y, v, dinv, b, w)


def _drug_fin(y, v, dinv, b, w, fin, fout):
    return pl.pallas_call(
        _drug_fin_body,
        grid=(NDP // ROWB,),
        in_specs=[_row_spec(fin), _row_spec(fin), pl.BlockSpec((ROWB,), lambda i: (i,)),
                  _vec_spec(fout), _full_spec(fin, fout)],
        out_specs=_row_spec(fout),
        out_shape=jax.ShapeDtypeStruct((NDP, fout), F32),
    )(y, v, dinv, b, w)


RPT = 208
ACCR = RPT + 8
EBLK = 1024
GBLK = 32
CAPL = 128
MCAP = 16 * CAPL + 2 * GBLK


def _sc_fire(u_hbm, msrc, b, buf, sem):
    pltpu.async_copy(u_hbm.at[msrc.at[pl.ds(b * GBLK, GBLK)]], buf, sem)


def _sc_wait(u_hbm, buf, sem):
    pltpu.make_async_copy(u_hbm.at[pl.ds(0, GBLK)], buf, sem).wait()


def _sc_agg_body(ne, f, u_hbm, sp_hbm, dp_hbm, out_hbm,
                 sblk, dblk, pend_s, pend_l, msrc, mloc,
                 rows_a, rows_b, acc, sem_a, sem_b):
    w = lax.axis_index("c") * 16 + lax.axis_index("s")
    base = w * RPT
    iota = lax.iota(jnp.int32, 16)
    lane_base = iota * CAPL

    pltpu.async_copy(sp_hbm, sblk, sem_a)
    pltpu.async_copy(dp_hbm, dblk, sem_b)

    zvec = jnp.zeros((16,), F32)

    def zacc(m, carry):
        for k in range(16):
            acc[pl.ds(m * 256 + 16 * k, 16)] = zvec
        return carry

    lax.fori_loop(0, ACCR * f // 256, zacc, 0)
    pltpu.make_async_copy(sp_hbm, sblk, sem_a).wait()
    pltpu.make_async_copy(dp_hbm, dblk, sem_b).wait()

    def scan(j, cnt):
        dv = dblk[pl.ds(16 * j, 16)]
        loc = dv - base
        ok = (loc >= 0) & (loc < RPT)
        idx = lane_base + jnp.minimum(cnt, CAPL - 1)
        plsc.store_scatter(pend_s, [idx], sblk[pl.ds(16 * j, 16)], mask=ok)
        plsc.store_scatter(pend_l, [idx], loc, mask=ok)
        return cnt + jnp.where(ok, 1, 0)

    cnt = lax.fori_loop(0, ne // 16, scan, jnp.zeros((16,), jnp.int32))

    pfx = plsc.cumsum(cnt)
    total = pfx[15]

    for l in range(16):
        nl = cnt[l]
        st = pfx[l] - nl

        def mv(tt, carry, l=l, nl=nl, st=st):
            pos = 16 * tt + iota
            m = pos < nl
            sv = pend_s[pl.ds(l * CAPL + 16 * tt, 16)]
            lv = pend_l[pl.ds(l * CAPL + 16 * tt, 16)]
            plsc.store_scatter(msrc, [st + pos], sv, mask=m)
            plsc.store_scatter(mloc, [st + pos], lv, mask=m)
            return carry

        lax.fori_loop(0, (nl + 15) // 16, mv, 0)

    for m in range(GBLK // 16):
        plsc.store_scatter(msrc, [total + 16 * m + iota],
                           jnp.zeros((16,), jnp.int32))
        plsc.store_scatter(mloc, [total + 16 * m + iota],
                           jnp.full((16,), RPT, jnp.int32))

    nb = total // GBLK + 1

    def accum(b, buf):
        for q in range(GBLK // 16):
            ldvec = mloc[pl.ds(b * GBLK + 16 * q, 16)]
            for r in range(16):
                ld = ldvec[r]
                for k in range(f // 16):
                    plsc.addupdate(acc.at[pl.ds(ld * f + 16 * k, 16)],
                                   buf[16 * q + r, pl.ds(16 * k, 16)])

    _sc_fire(u_hbm, msrc, 0, rows_a, sem_a)

    def pairs(i, carry):
        b0 = 2 * i

        @pl.when(b0 + 1 < nb)
        def _():
            _sc_fire(u_hbm, msrc, b0 + 1, rows_b, sem_b)
        _sc_wait(u_hbm, rows_a, sem_a)
        accum(b0, rows_a)

        @pl.when(b0 + 2 < nb)
        def _():
            _sc_fire(u_hbm, msrc, b0 + 2, rows_a, sem_a)

        @pl.when(b0 + 1 < nb)
        def _():
            _sc_wait(u_hbm, rows_b, sem_b)
            accum(b0 + 1, rows_b)
        return carry

    lax.fori_loop(0, (nb + 1) // 2, pairs, 0)

    pltpu.sync_copy(acc.at[pl.ds(0, RPT * f)],
                    out_hbm.at[pl.ds(base * f, RPT * f)])


def _sc_agg(u, sp, dp, f):
    ne = sp.shape[0]
    mesh = plsc.VectorSubcoreMesh(core_axis_name="c", subcore_axis_name="s")
    k = pl.kernel(
        functools.partial(_sc_agg_body, ne, f),
        out_type=jax.ShapeDtypeStruct((NDP * f,), F32),
        mesh=mesh,
        compiler_params=pltpu.CompilerParams(needs_layout_passes=False),
        scratch_types=[
            pltpu.VMEM((ne,), jnp.int32),
            pltpu.VMEM((ne,), jnp.int32),
            pltpu.VMEM((16 * CAPL,), jnp.int32),
            pltpu.VMEM((16 * CAPL,), jnp.int32),
            pltpu.VMEM((MCAP,), jnp.int32),
            pltpu.VMEM((MCAP,), jnp.int32),
            pltpu.VMEM((GBLK, f), F32),
            pltpu.VMEM((GBLK, f), F32),
            pltpu.VMEM((ACCR * f,), F32),
            pltpu.SemaphoreType.DMA,
            pltpu.SemaphoreType.DMA,
        ],
    )
    return k(u, sp, dp).reshape(NDP, f)


def _edge_layout(s, d):
    e = s.shape[0]
    tot = -(-e // EBLK) * EBLK
    sp = jnp.zeros((tot,), jnp.int32).at[:e].set(s)
    dp = jnp.full((tot,), NDP, jnp.int32).at[:e].set(d)
    return sp, dp


SEGR = 224
SEGF = 384


def _sx_fire(x_hbm, w, m, buf, sem):
    pltpu.async_copy(x_hbm.at[pl.ds(w * RPT + 16 * m, 16)], buf, sem)


def _sx_wait(x_hbm, buf, sem):
    pltpu.make_async_copy(x_hbm.at[pl.ds(0, 16)], buf, sem).wait()


def _sc_segmax_body(x_hbm, b_hbm, out_hbm, bb, rb_a, rb_b, acc,
                    sem_a, sem_b, sem_c):
    w = lax.axis_index("c") * 16 + lax.axis_index("s")
    ninf = jnp.full((16,), -jnp.inf, F32)
    nchunk = RPT // 16

    _sx_fire(x_hbm, w, 0, rb_a, sem_a)
    pltpu.async_copy(b_hbm.at[pl.ds(w * RPT, RPT)], bb, sem_c)

    def zacc(m, carry):
        for k in range(16):
            acc[pl.ds(m * 256 + 16 * k, 16)] = ninf
        return carry

    lax.fori_loop(0, (SEGR * SEGF + 256) // 256, zacc, 0)
    pltpu.make_async_copy(b_hbm.at[pl.ds(0, RPT)], bb, sem_c).wait()

    def chunk(m, buf):
        bv = bb[pl.ds(16 * m, 16)]
        for r in range(16):
            sb = bv[r] * SEGF
            for k in range(SEGF // 16):
                cur = acc[pl.ds(sb + 16 * k, 16)]
                acc[pl.ds(sb + 16 * k, 16)] = jnp.maximum(
                    cur, buf[r, pl.ds(16 * k, 16)])

    def pairs(i, carry):
        m0 = 2 * i

        @pl.when(m0 + 1 < nchunk)
        def _():
            _sx_fire(x_hbm, w, m0 + 1, rb_b, sem_b)
        _sx_wait(x_hbm, rb_a, sem_a)
        chunk(m0, rb_a)

        @pl.when(m0 + 2 < nchunk)
        def _():
            _sx_fire(x_hbm, w, m0 + 2, rb_a, sem_a)

        @pl.when(m0 + 1 < nchunk)
        def _():
            _sx_wait(x_hbm, rb_b, sem_b)
            chunk(m0 + 1, rb_b)
        return carry

    lax.fori_loop(0, (nchunk + 1) // 2, pairs, 0)

    pltpu.sync_copy(acc.at[pl.ds(0, SEGR * SEGF)],
                    out_hbm.at[pl.ds(w * SEGR * SEGF, SEGR * SEGF)])


def _sc_segmax(x4, batch_pad):
    mesh = plsc.VectorSubcoreMesh(core_axis_name="c", subcore_axis_name="s")
    k = pl.kernel(
        _sc_segmax_body,
        out_type=jax.ShapeDtypeStruct((32 * SEGR * SEGF,), F32),
        mesh=mesh,
        compiler_params=pltpu.CompilerParams(needs_layout_passes=False),
        scratch_types=[
            pltpu.VMEM((RPT,), jnp.int32),
            pltpu.VMEM((16, SEGF), F32),
            pltpu.VMEM((16, SEGF), F32),
            pltpu.VMEM((SEGR * SEGF + 256,), F32),
            pltpu.SemaphoreType.DMA,
            pltpu.SemaphoreType.DMA,
            pltpu.SemaphoreType.DMA,
        ],
    )
    return k(x4, batch_pad).reshape(32, SEGR, SEGF)


CROWS = 16


def _sc_counts_merge(pend, merged, cnt, dump, iota):
    pfx = plsc.cumsum(cnt)
    total = pfx[15]
    for l in range(16):
        nl = cnt[l]
        st = pfx[l] - nl

        def mv(tt, carry, l=l, nl=nl, st=st):
            pos = 16 * tt + iota
            m = pos < nl
            lv = pend[pl.ds(l * CAPL + 16 * tt, 16)]
            plsc.store_scatter(merged, [st + pos], lv, mask=m)
            return carry

        lax.fori_loop(0, (nl + 15) // 16, mv, 0)
    plsc.store_scatter(merged, [total + iota],
                       jnp.full((16,), dump, jnp.int32))
    return total


def _sc_counts_apply(merged, total, acc, e0):
    def app(t, carry):
        ldvec = merged[pl.ds(16 * t, 16)]
        for r in range(16):
            plsc.addupdate(acc.at[pl.ds(ldvec[r], 16)], e0)
        return carry

    lax.fori_loop(0, total // 16 + 1, app, 0)


def _sc_counts_body(nd, ne, dd_hbm, es_hbm, ed_hbm, deg_hbm, c_hbm,
                    stage, stage2, pend, merged, dacc, cacc, sem):
    w = lax.axis_index("c") * 16 + lax.axis_index("s")
    iota = lax.iota(jnp.int32, 16)
    lane_base = iota * CAPL
    zvec = jnp.zeros((16,), F32)
    e0 = jnp.where(iota == 0, 1.0, 0.0).astype(F32)

    pltpu.async_copy(dd_hbm, stage.at[pl.ds(0, nd)], sem)
    pltpu.async_copy(es_hbm, stage2.at[pl.ds(0, ne)], sem)

    for k in range(RPT // 16 + 1):
        dacc[pl.ds(16 * k, 16)] = zvec

    def zc(m, carry):
        for k in range(16):
            cacc[pl.ds(m * 256 + 16 * k, 16)] = zvec
        return carry

    lax.fori_loop(0, (CROWS * 512 + 256) // 256, zc, 0)
    pltpu.make_async_copy(dd_hbm, stage.at[pl.ds(0, nd)], sem).wait()
    pltpu.make_async_copy(es_hbm, stage2.at[pl.ds(0, ne)], sem).wait()

    base = w * RPT

    def scan1(j, cnt):
        loc = stage[pl.ds(16 * j, 16)] - base
        ok = (loc >= 0) & (loc < RPT)
        idx = lane_base + jnp.minimum(cnt, CAPL - 1)
        plsc.store_scatter(pend, [idx], loc, mask=ok)
        return cnt + jnp.where(ok, 1, 0)

    cnt = lax.fori_loop(0, nd // 16, scan1, jnp.zeros((16,), jnp.int32))
    total = _sc_counts_merge(pend, merged, cnt, RPT, iota)
    _sc_counts_apply(merged, total, dacc, e0)
    pltpu.sync_copy(dacc.at[pl.ds(0, RPT)], deg_hbm.at[pl.ds(base, RPT)])

    pltpu.sync_copy(ed_hbm, stage.at[pl.ds(0, ne)])
    cbase = w * CROWS

    def scan2(j, cnt2):
        loc = stage[pl.ds(16 * j, 16)] - cbase
        ok = (loc >= 0) & (loc < CROWS)
        sv = stage2[pl.ds(16 * j, 16)]
        idx = lane_base + jnp.minimum(cnt2, CAPL - 1)
        plsc.store_scatter(pend, [idx], loc * 512 + sv, mask=ok)
        return cnt2 + jnp.where(ok, 1, 0)

    cnt2 = lax.fori_loop(0, ne // 16, scan2, jnp.zeros((16,), jnp.int32))
    total2 = _sc_counts_merge(pend, merged, cnt2, CROWS * 512, iota)
    _sc_counts_apply(merged, total2, cacc, e0)
    pltpu.sync_copy(cacc.at[pl.ds(0, CROWS * 512)],
                    c_hbm.at[pl.ds(cbase * 512, CROWS * 512)])


def _sc_counts(dd, es, ed):
    nd = dd.shape[0]
    ne = es.shape[0]
    mesh = plsc.VectorSubcoreMesh(core_axis_name="c", subcore_axis_name="s")
    k = pl.kernel(
        functools.partial(_sc_counts_body, nd, ne),
        out_type=[jax.ShapeDtypeStruct((NDP,), F32),
                  jax.ShapeDtypeStruct((512 * 512,), F32)],
        mesh=mesh,
        compiler_params=pltpu.CompilerParams(needs_layout_passes=False),
        scratch_types=[
            pltpu.VMEM((ne,), jnp.int32),
            pltpu.VMEM((ne,), jnp.int32),
            pltpu.VMEM((16 * CAPL,), jnp.int32),
            pltpu.VMEM((16 * CAPL + 2 * GBLK,), jnp.int32),
            pltpu.VMEM((RPT + 32,), F32),
            pltpu.VMEM((CROWS * 512 + 256 + 32,), F32),
            pltpu.SemaphoreType.DMA,
        ],
    )
    deg, cf = k(dd, es, ed)
    return deg, cf.reshape(512, 512)


def _middle_body(pool_ref, x1_ref, c_ref, wfc_ref, bfc_ref,
                 wg_ref, bg_ref, sel_ref, fc1w_ref, fc1b_ref,
                 w0t_ref, w0b_ref, b0_ref, a_ref, b_ref):
    rr = jax.lax.broadcasted_iota(jnp.int32, (512, 512), 0)
    cc = jax.lax.broadcasted_iota(jnp.int32, (512, 512), 1)
    C = c_ref[...] + jnp.where((rr == cc) & (rr < N_NODES), 1.0, 0.0)
    deg = jnp.sum(C, axis=1)
    dinv = jnp.where(deg > 0, jax.lax.rsqrt(deg), 0.0)
    Adj = dinv[:, None] * C * dinv[None, :]

    pmax = jnp.max(pool_ref[...], axis=0)
    pooled = jnp.where(jnp.isfinite(pmax), pmax, 0.0)
    gfeat = jax.nn.relu(_bdot(pooled, wfc_ref[...])
                        + bfc_ref[...][None, :])
    rows = jax.lax.broadcasted_iota(jnp.int32, (512, 1), 0)
    xcat = x1_ref[...] + jnp.where(rows < N_DRUGS, _pad_rows(gfeat, 512), 0.0)

    sel = sel_ref[...][:, None]
    xsel = jnp.zeros((512, 512), F32)
    for l in range(3):
        xl = jax.nn.relu(_bdot(Adj.astype(F32), _bdot(xcat, wg_ref[l]))
                         + bg_ref[l][None, :])
        xsel = xsel + jnp.where(sel == l, xl, 0.0)
    xf = jax.nn.relu(_bdot(xsel, fc1w_ref[...]) + fc1b_ref[...][None, :])
    a_ref[...] = (_bdot(xf, w0t_ref[0]) + _bdot(xcat, w0t_ref[1])
                  + b0_ref[...][None, :])
    b_ref[...] = _bdot(xf, w0b_ref[0]) + _bdot(xcat, w0b_ref[1])


def _pad_rows(a, n):
    return jnp.pad(a, ((0, n - a.shape[0]), (0, 0)))


def _middle(pool, x1p, C, wfc, bfc, wg, bg, sel, fc1w, fc1b, w0t, w0b, b0):
    fs = _full_spec
    return pl.pallas_call(
        _middle_body,
        grid=(1,),
        in_specs=[pl.BlockSpec((32, 224, 384), lambda i: (0, 0, 0)),
                  fs(512, 512), fs(512, 512), fs(384, 512),
                  _vec_spec(512), pl.BlockSpec((3, 512, 512), lambda i: (0, 0, 0)),
                  pl.BlockSpec((3, 512), lambda i: (0, 0)),
                  pl.BlockSpec((512,), lambda i: (0,)), fs(512, 512),
                  _vec_spec(512), pl.BlockSpec((2, 512, 512), lambda i: (0, 0, 0)),
                  pl.BlockSpec((2, 512, 512), lambda i: (0, 0, 0)),
                  _vec_spec(512)],
        out_specs=[fs(512, 512), fs(512, 512)],
        out_shape=[jax.ShapeDtypeStruct((512, 512), F32),
                   jax.ShapeDtypeStruct((512, 512), F32)],
    )(pool, x1p, C, wfc, bfc, wg, bg, sel, fc1w, fc1b, w0t, w0b, b0)


def _mlp_tail(z0, w1_ref, b1_ref, w2_ref, b2_ref, wl_ref, bl_ref):
    h = jax.nn.relu(z0)
    h = jax.nn.relu(_bdot(h, w1_ref[...]) + b1_ref[...][None, :])
    h = jax.nn.relu(_bdot(h, w2_ref[...]) + b2_ref[...][None, :])
    logit = jnp.sum(h * wl_ref[...][None, :], axis=1) + bl_ref[0]
    return jax.nn.sigmoid(logit)


def _out2_body(a2_ref, b2_ref, w1_ref, b1_ref, w2_ref, b2w_ref,
               wl_ref, bl_ref, o_ref, *, bi):
    z0 = (b2_ref[...][:, None, :] + a2_ref[...][None, :, :]
          ).reshape(bi * 272, 512)
    o_ref[...] = _mlp_tail(z0, w1_ref, b1_ref, w2_ref, b2w_ref,
                           wl_ref, bl_ref).reshape(bi, 272)


def _out2(a2, b2, w1, b1, w2, b2w, wl, bl, bi=16):
    nblk = 224 // bi
    return pl.pallas_call(
        functools.partial(_out2_body, bi=bi),
        grid=(nblk,),
        in_specs=[pl.BlockSpec((272, 512), lambda i: (0, 0)),
                  pl.BlockSpec((bi, 512), lambda i: (i, 0)),
                  _full_spec(512, 512), _vec_spec(512),
                  _full_spec(512, 512), _vec_spec(512), _vec_spec(512),
                  _vec_spec(8)],
        out_specs=pl.BlockSpec((bi, 272), lambda i: (i, 0)),
        out_shape=jax.ShapeDtypeStruct((224, 272), F32),
    )(a2, b2, w1, b1, w2, b2w, wl, bl)


def _out1_body(rr_ref, dd_ref, a_ref, b_ref,
               w1_ref, b1_ref, w2_ref, b2_ref, wl_ref, bl_ref, o_ref):
    cols = jax.lax.broadcasted_iota(jnp.int32, (1024, 512), 1)
    ohr = (rr_ref[...][:, None] == cols).astype(F32)
    ohd = (dd_ref[...][:, None] == cols).astype(F32)
    z0 = _bdot(ohr, a_ref[...]) + _bdot(ohd, b_ref[...])
    o_ref[...] = _mlp_tail(z0, w1_ref, b1_ref, w2_ref, b2_ref,
                           wl_ref, bl_ref)


def _out1(rr, ddx, a, b, w1, b1, w2, b2, wl, bl):
    return pl.pallas_call(
        _out1_body,
        grid=(8,),
        in_specs=[pl.BlockSpec((1024,), lambda i: (i,)),
                  pl.BlockSpec((1024,), lambda i: (i,)),
                  _full_spec(512, 512), _full_spec(512, 512),
                  _full_spec(512, 512), _vec_spec(512), _full_spec(512, 512),
                  _vec_spec(512), _vec_spec(512), _vec_spec(8)],
        out_specs=pl.BlockSpec((1024,), lambda i: (i,)),
        out_shape=jax.ShapeDtypeStruct((8192,), F32),
    )(rr, ddx, a, b, w1, b1, w2, b2, wl, bl)


def kernel(x1, edges, hop, edges2, drug_x, drug_edge_index, drug_batch, params):
    p = params
    s, dd = drug_edge_index[0], drug_edge_index[1]

    inv = 1.0 / np.sqrt(1.0 + BN_EPS)
    g0, g1, g2 = p['bn_g0'] * inv, p['bn_g1'] * inv, p['bn_g2'] * inv
    w1p = _pad2(g0[:, None] * p['d_W1'], 512, 512)
    b1p = _pad1(p['bn_b0'] @ p['d_W1'] + p['d_b1'], 512)
    w2p = _pad2(g1[:, None] * p['d_W2'], 512, 512)
    b2p = _pad1(p['bn_b1'] @ p['d_W2'] + p['d_b2'], 512)
    wlp = _pad1((g2[:, None] * p['d_Wl'])[:, 0], 512)
    blp = _pad1(p['bn_b2'] @ p['d_Wl'] + p['d_bl'], 8)
    b0p = _pad1(p['d_b0'], 512)

    gw1 = _pad2(p['g_W1'], 128, 128)
    gw2 = _pad2(p['g_W2'], 128, 256)
    gw3 = _pad2(p['g_W3'], 256, 384)
    gwfc = _pad2(p['g_Wfc'], 384, 512)
    gb1 = _pad1(p['g_b1'], 128)
    gb2 = _pad1(p['g_b2'], 256)
    gb3 = _pad1(p['g_b3'], 384)
    gbfc = _pad1(p['g_bfc'], 512)
    wg = jnp.stack([_pad2(p['W_g%d' % l], 512, 512) for l in range(3)])
    bg = jnp.stack([_pad1(p['b_g%d' % l], 512) for l in range(3)])
    fc1w = _pad2(p['fc1_W'], 512, 512)
    fc1b = _pad1(p['fc1_b'], 512)
    w0t = jnp.stack([_pad2(p['d_W0'][:489], 512, 512),
                     _pad2(p['d_W0'][489:978], 512, 512)])
    w0b = jnp.stack([_pad2(p['d_W0'][978:978 + 489], 512, 512),
                     _pad2(p['d_W0'][978 + 489:], 512, 512)])

    spi, dpi = _edge_layout(s, dd)
    esp, edp = _edge_layout(edges[0], edges[1])
    deg, C = _sc_counts(dpi, esp, edp)
    dinv = jnp.where(jnp.arange(NDP) < ND, (deg + 1.0) ** -0.5, 0.0)

    xq = _pad2(drug_x, NDP, 128)
    v1 = _drug_scale(xq, dinv, 128)
    y1 = _sc_agg(v1, spi, dpi, 128)
    v2, _ = _drug_mid(y1, v1, dinv, gb1, gw1, 128, 128)
    y2 = _sc_agg(v2, spi, dpi, 128)
    v3, _ = _drug_mid(y2, v2, dinv, gb2, gw2, 128, 256)
    y3 = _sc_agg(v3, spi, dpi, 256)
    x4 = _drug_fin(y3, v3, dinv, gb3, gw3, 256, 384)

    batch_pad = jnp.full((NDP,), N_DRUGS, jnp.int32).at[:ND].set(drug_batch)
    pool = _sc_segmax(x4, batch_pad)

    x1p = _pad2(x1, 512, 512)
    sel = _pad1(jnp.where(hop == 0, 2, hop - 1).astype(jnp.int32), 512)
    A, B = _middle(pool, x1p, C, gwfc, gbfc, wg, bg, sel, fc1w, fc1b,
                   w0t, w0b, b0p)

    a2 = _pad_rows(A[N_DRUGS:N_NODES], 272).astype(BF16)
    b2 = B[:224].astype(BF16)
    out2 = _out2(a2, b2, w1p, b1p, w2p, b2p, wlp, blp)[:N_DRUGS, :271]

    out1 = _out1(edges2[1], edges2[0], A, B,
                 w1p, b1p, w2p, b2p, wlp, blp)

    return out1, out2
